# Initial kernel scaffold; baseline (speedup 1.0000x reference)
import jax
import jax.numpy as jnp
from jax.experimental import pallas as pl


def kernel(user_table, item_table, edge_index):
    raise NotImplementedError("write your pallas kernel here")


# speedup vs baseline: 4.0370x; 4.0370x over previous
import functools

import jax
import jax.numpy as jnp
from jax import lax
from jax.experimental import pallas as pl
from jax.experimental.pallas import tpu as pltpu
from jax.experimental.pallas import tpu_sc as plsc

N_U = 5000
N_I = 5000
N_NODES = 10000
D = 128
DH = 64
E = 320000
NUM_LAYERS = 3

N_TILES = 16
RPT = N_NODES // N_TILES
RCH = 125
N_RCH = RPT // RCH
EPT = E // N_TILES
ECH = 80
N_ECH = EPT // ECH
NSEG = DH // 16

_ZCH = [ECH] * (RPT // ECH) + ([RPT % ECH] if RPT % ECH else [])


def _gcn_body(user_hbm, item_hbm, edge_hbm, out_hbm,
              a_sh, b_sh, deg_sh,
              xbuf, hbuf, gbuf, rbuf, sidx, didx, ones, sem):
  c = lax.axis_index("c")
  s = lax.axis_index("s")
  col0 = c * DH
  r0 = s * RPT
  e0 = s * EPT

  zv = jnp.zeros((16,), jnp.float32)

  def fill_gbuf_zero():
    def body(i, _):
      gbuf[i // NSEG, pl.ds((i % NSEG) * 16, 16)] = zv
      return 0
    lax.fori_loop(0, ECH * NSEG, body, 0)

  def zero_rows(dst_sh):
    off = 0
    for n in _ZCH:
      pltpu.sync_copy(gbuf.at[pl.ds(0, n), :],
                      dst_sh.at[pl.ds(r0 + off, n), :])
      off += n

  def _ones_fill(i, _):
    ones[i, :] = zv + 1.0
    return 0
  lax.fori_loop(0, ECH, _ones_fill, 0)

  def _rbuf_fill(i, _):
    rbuf[i, :] = zv
    return 0
  lax.fori_loop(0, RCH, _rbuf_fill, 0)

  fill_gbuf_zero()
  zero_rows(b_sh)
  for k in range(N_RCH):
    pltpu.sync_copy(rbuf, deg_sh.at[pl.ds(r0 + k * RCH, RCH), :])

  @pl.when(s < 8)
  def _():
    pltpu.sync_copy(user_hbm.at[pl.ds(r0, RPT), pl.ds(col0, DH)],
                    a_sh.at[pl.ds(r0, RPT), :])

  @pl.when(s >= 8)
  def _():
    pltpu.sync_copy(item_hbm.at[pl.ds(r0 - N_U, RPT), pl.ds(col0, DH)],
                    a_sh.at[pl.ds(r0, RPT), :])

  for k in range(N_RCH):
    pltpu.sync_copy(a_sh.at[pl.ds(r0 + k * RCH, RCH), :], xbuf)
    pltpu.sync_copy(xbuf,
                    out_hbm.at[pl.ds(r0 + k * RCH, RCH), pl.ds(col0, DH)])

  plsc.subcore_barrier()

  def edge_phase(in_sh, out_sh, count_deg):
    def body(k, _):
      base = e0 + k * ECH
      pltpu.sync_copy(edge_hbm.at[0, pl.ds(base, ECH)], sidx.at[0])
      pltpu.sync_copy(edge_hbm.at[1, pl.ds(base, ECH)], didx.at[0])
      pltpu.async_copy(in_sh.at[sidx.at[0]], gbuf, sem).wait()
      pltpu.sync_copy(gbuf, out_sh.at[didx.at[0]], add=True)
      if count_deg:
        pltpu.sync_copy(ones, deg_sh.at[didx.at[0]], add=True)
      return 0
    lax.fori_loop(0, N_ECH, body, 0)

  def recip_phase():
    for k in range(N_RCH):
      ro = r0 + k * RCH
      pltpu.sync_copy(deg_sh.at[pl.ds(ro, RCH), :], rbuf)
      def body(i, _):
        rbuf[i, :] = 1.0 / jnp.maximum(rbuf[i, :], 1.0)
        return 0
      lax.fori_loop(0, RCH, body, 0)
      pltpu.sync_copy(rbuf, deg_sh.at[pl.ds(ro, RCH), :])

  def norm_phase(out_sh, znext_sh, last):
    if znext_sh is not None:
      fill_gbuf_zero()
      zero_rows(znext_sh)
    for k in range(N_RCH):
      ro = r0 + k * RCH
      pltpu.sync_copy(out_sh.at[pl.ds(ro, RCH), :], xbuf)
      pltpu.sync_copy(deg_sh.at[pl.ds(ro, RCH), :], rbuf)
      pltpu.sync_copy(out_hbm.at[pl.ds(ro, RCH), pl.ds(col0, DH)], hbuf)
      def body(i, _):
        rec = rbuf[i, :]
        for j in range(NSEG):
          sl = pl.ds(j * 16, 16)
          x = xbuf[i, sl] * rec
          xbuf[i, sl] = x
          if last:
            hbuf[i, sl] = (hbuf[i, sl] + x) * 0.25
          else:
            hbuf[i, sl] = hbuf[i, sl] + x
        return 0
      lax.fori_loop(0, RCH, body, 0)
      if not last:
        pltpu.sync_copy(xbuf, out_sh.at[pl.ds(ro, RCH), :])
      pltpu.sync_copy(hbuf, out_hbm.at[pl.ds(ro, RCH), pl.ds(col0, DH)])

  edge_phase(a_sh, b_sh, True)
  plsc.subcore_barrier()
  recip_phase()
  norm_phase(b_sh, a_sh, False)
  plsc.subcore_barrier()
  edge_phase(b_sh, a_sh, False)
  plsc.subcore_barrier()
  norm_phase(a_sh, b_sh, False)
  plsc.subcore_barrier()
  edge_phase(a_sh, b_sh, False)
  plsc.subcore_barrier()
  norm_phase(b_sh, None, True)


@functools.partial(
    pl.kernel,
    out_type=jax.ShapeDtypeStruct((N_NODES, D), jnp.float32),
    mesh=plsc.VectorSubcoreMesh(core_axis_name="c", subcore_axis_name="s"),
    compiler_params=pltpu.CompilerParams(use_tc_tiling_on_sc=False),
    scratch_types=[
        pltpu.VMEM_SHARED((N_NODES, DH), jnp.float32),
        pltpu.VMEM_SHARED((N_NODES, DH), jnp.float32),
        pltpu.VMEM_SHARED((N_NODES, 16), jnp.float32),
        pltpu.VMEM((RCH, DH), jnp.float32),
        pltpu.VMEM((RCH, DH), jnp.float32),
        pltpu.VMEM((ECH, DH), jnp.float32),
        pltpu.VMEM((RCH, 16), jnp.float32),
        pltpu.VMEM((1, ECH), jnp.int32),
        pltpu.VMEM((1, ECH), jnp.int32),
        pltpu.VMEM((ECH, 16), jnp.float32),
        pltpu.SemaphoreType.DMA,
    ],
)
def _gcn(user_hbm, item_hbm, edge_hbm, out_hbm, *scratch):
  _gcn_body(user_hbm, item_hbm, edge_hbm, out_hbm, *scratch)


@jax.jit
def kernel(user_table, item_table, edge_index):
  out = _gcn(user_table, item_table, edge_index)
  return out[:N_U], out[N_U:]


# speedup vs baseline: 8.6858x; 2.1515x over previous
import functools

import jax
import jax.numpy as jnp
from jax import lax
from jax.experimental import pallas as pl
from jax.experimental.pallas import tpu as pltpu
from jax.experimental.pallas import tpu_sc as plsc

N_U = 5000
N_I = 5000
N_NODES = 10000
D = 128
DH = 64
E = 320000
NUM_LAYERS = 3

N_TILES = 16
RPT = N_NODES // N_TILES
ECH = 128
NCHUNK = E // ECH
CPT = NCHUNK // N_TILES
BLK = 13
NBLK = CPT // BLK
NSEG = DH // 16

_RCH = [(o, min(80, RPT - o)) for o in range(0, RPT, 80)]


def _gcn_body(user_hbm, item_hbm, edge_hbm, out_hbm,
              a_sh, b_sh, deg_sh,
              xbuf, hbuf, gbuf, rbuf, sidx, didx, ones,
              gsem, ssem, isem, dsem):
  c = lax.axis_index("c")
  s = lax.axis_index("s")
  col0 = c * DH
  r0 = s * RPT
  c0 = s * CPT

  zv = jnp.zeros((16,), jnp.float32)

  def fill_gbuf0_zero():
    def body(i, _):
      gbuf[0, i // NSEG, pl.ds((i % NSEG) * 16, 16)] = zv
      return 0
    lax.fori_loop(0, ECH * NSEG, body, 0)

  def zero_rows(dst_sh):
    for off, n in _RCH:
      pltpu.sync_copy(gbuf.at[0, pl.ds(0, n), :],
                      dst_sh.at[pl.ds(r0 + off, n), :])

  def _ones_fill(i, _):
    ones[i, :] = zv + 1.0
    return 0
  lax.fori_loop(0, ECH, _ones_fill, 0)

  def _rbuf_fill(i, _):
    rbuf[i, :] = zv
    return 0
  lax.fori_loop(0, 80, _rbuf_fill, 0)

  fill_gbuf0_zero()
  zero_rows(b_sh)
  for off, n in _RCH:
    pltpu.sync_copy(rbuf.at[pl.ds(0, n), :],
                    deg_sh.at[pl.ds(r0 + off, n), :])

  @pl.when(s < 8)
  def _():
    pltpu.sync_copy(user_hbm.at[pl.ds(r0, RPT), pl.ds(col0, DH)],
                    a_sh.at[pl.ds(r0, RPT), :])

  @pl.when(s >= 8)
  def _():
    pltpu.sync_copy(item_hbm.at[pl.ds(r0 - N_U, RPT), pl.ds(col0, DH)],
                    a_sh.at[pl.ds(r0, RPT), :])

  for off, n in _RCH:
    pltpu.sync_copy(a_sh.at[pl.ds(r0 + off, n), :], xbuf.at[pl.ds(0, n), :])
    pltpu.sync_copy(xbuf.at[pl.ds(0, n), :],
                    out_hbm.at[pl.ds(r0 + off, n), pl.ds(col0, DH)])

  plsc.subcore_barrier()

  def edge_phase(in_sh, out_sh, count_deg):
    pltpu.sync_copy(edge_hbm.at[0, pl.ds(c0, BLK), :], sidx.at[0])
    pltpu.sync_copy(edge_hbm.at[1, pl.ds(c0, BLK), :], didx.at[0])

    def block(k, _):
      p = lax.rem(k, 2)
      pn = lax.rem(k + 1, 2)

      @pl.when(k + 1 < NBLK)
      def _():
        nb = c0 + BLK * (k + 1)
        pltpu.async_copy(edge_hbm.at[0, pl.ds(nb, BLK), :], sidx.at[pn], isem)
        pltpu.async_copy(edge_hbm.at[1, pl.ds(nb, BLK), :], didx.at[pn], isem)

      pltpu.async_copy(in_sh.at[sidx.at[p, 0]], gbuf.at[0], gsem)
      for j in range(BLK):
        g = j % 2
        pltpu.make_async_copy(in_sh.at[sidx.at[p, 0]],
                              gbuf.at[g], gsem).wait()
        if j + 1 < BLK:
          if j >= 1:
            pltpu.make_async_copy(gbuf.at[(j + 1) % 2],
                                  out_sh.at[didx.at[p, 0]], ssem).wait()
          pltpu.async_copy(in_sh.at[sidx.at[p, j + 1]],
                           gbuf.at[(j + 1) % 2], gsem)
        pltpu.async_copy(gbuf.at[g], out_sh.at[didx.at[p, j]], ssem,
                         add=True)
        if count_deg:
          pltpu.async_copy(ones, deg_sh.at[didx.at[p, j]], dsem, add=True)

      pltpu.make_async_copy(gbuf.at[0], out_sh.at[didx.at[p, 0]],
                            ssem).wait()
      pltpu.make_async_copy(gbuf.at[1], out_sh.at[didx.at[p, 0]],
                            ssem).wait()
      if count_deg:
        for j in range(BLK):
          pltpu.make_async_copy(ones, deg_sh.at[didx.at[p, 0]], dsem).wait()

      @pl.when(k + 1 < NBLK)
      def _():
        pltpu.make_async_copy(edge_hbm.at[0, pl.ds(c0, BLK), :],
                              sidx.at[pn], isem).wait()
        pltpu.make_async_copy(edge_hbm.at[1, pl.ds(c0, BLK), :],
                              didx.at[pn], isem).wait()
      return 0

    lax.fori_loop(0, NBLK, block, 0)

    @pl.when(s < 4)
    def _():
      tb = N_TILES * CPT + s
      pltpu.sync_copy(edge_hbm.at[0, pl.ds(tb, 1), :], sidx.at[0, pl.ds(0, 1)])
      pltpu.sync_copy(edge_hbm.at[1, pl.ds(tb, 1), :], didx.at[0, pl.ds(0, 1)])
      pltpu.async_copy(in_sh.at[sidx.at[0, 0]], gbuf.at[0], gsem)
      pltpu.make_async_copy(in_sh.at[sidx.at[0, 0]], gbuf.at[0], gsem).wait()
      pltpu.sync_copy(gbuf.at[0], out_sh.at[didx.at[0, 0]], add=True)
      if count_deg:
        pltpu.sync_copy(ones, deg_sh.at[didx.at[0, 0]], add=True)

  def recip_phase():
    for off, n in _RCH:
      ro = r0 + off
      pltpu.sync_copy(deg_sh.at[pl.ds(ro, n), :], rbuf.at[pl.ds(0, n), :])
      def body(i, _):
        rbuf[i, :] = 1.0 / jnp.maximum(rbuf[i, :], 1.0)
        return 0
      lax.fori_loop(0, n, body, 0)
      pltpu.sync_copy(rbuf.at[pl.ds(0, n), :], deg_sh.at[pl.ds(ro, n), :])

  def norm_phase(out_sh, znext_sh, last):
    if znext_sh is not None:
      fill_gbuf0_zero()
      zero_rows(znext_sh)
    for off, n in _RCH:
      ro = r0 + off
      pltpu.sync_copy(out_sh.at[pl.ds(ro, n), :], xbuf.at[pl.ds(0, n), :])
      pltpu.sync_copy(deg_sh.at[pl.ds(ro, n), :], rbuf.at[pl.ds(0, n), :])
      pltpu.sync_copy(out_hbm.at[pl.ds(ro, n), pl.ds(col0, DH)],
                      hbuf.at[pl.ds(0, n), :])
      def body(i, _):
        rec = rbuf[i, :]
        for j in range(NSEG):
          sl = pl.ds(j * 16, 16)
          x = xbuf[i, sl] * rec
          xbuf[i, sl] = x
          if last:
            hbuf[i, sl] = (hbuf[i, sl] + x) * 0.25
          else:
            hbuf[i, sl] = hbuf[i, sl] + x
        return 0
      lax.fori_loop(0, n, body, 0)
      if not last:
        pltpu.sync_copy(xbuf.at[pl.ds(0, n), :], out_sh.at[pl.ds(ro, n), :])
      pltpu.sync_copy(hbuf.at[pl.ds(0, n), :],
                      out_hbm.at[pl.ds(ro, n), pl.ds(col0, DH)])

  edge_phase(a_sh, b_sh, True)
  plsc.subcore_barrier()
  recip_phase()
  norm_phase(b_sh, a_sh, False)
  plsc.subcore_barrier()
  edge_phase(b_sh, a_sh, False)
  plsc.subcore_barrier()
  norm_phase(a_sh, b_sh, False)
  plsc.subcore_barrier()
  edge_phase(a_sh, b_sh, False)
  plsc.subcore_barrier()
  norm_phase(b_sh, None, True)


@functools.partial(
    pl.kernel,
    out_type=jax.ShapeDtypeStruct((N_NODES, D), jnp.float32),
    mesh=plsc.VectorSubcoreMesh(core_axis_name="c", subcore_axis_name="s"),
    compiler_params=pltpu.CompilerParams(use_tc_tiling_on_sc=False),
    scratch_types=[
        pltpu.VMEM_SHARED((N_NODES, DH), jnp.float32),
        pltpu.VMEM_SHARED((N_NODES, DH), jnp.float32),
        pltpu.VMEM_SHARED((N_NODES, 16), jnp.float32),
        pltpu.VMEM((80, DH), jnp.float32),
        pltpu.VMEM((80, DH), jnp.float32),
        pltpu.VMEM((2, ECH, DH), jnp.float32),
        pltpu.VMEM((80, 16), jnp.float32),
        pltpu.VMEM((2, BLK, ECH), jnp.int32),
        pltpu.VMEM((2, BLK, ECH), jnp.int32),
        pltpu.VMEM((ECH, 16), jnp.float32),
        pltpu.SemaphoreType.DMA,
        pltpu.SemaphoreType.DMA,
        pltpu.SemaphoreType.DMA,
        pltpu.SemaphoreType.DMA,
    ],
)
def _gcn(user_hbm, item_hbm, edge_hbm, out_hbm, *scratch):
  _gcn_body(user_hbm, item_hbm, edge_hbm, out_hbm, *scratch)


@jax.jit
def kernel(user_table, item_table, edge_index):
  out = _gcn(user_table, item_table, edge_index.reshape(2, NCHUNK, ECH))
  return out[:N_U], out[N_U:]


# speedup vs baseline: 12.2224x; 1.4072x over previous
import functools

import jax
import jax.numpy as jnp
from jax import lax
from jax.experimental import pallas as pl
from jax.experimental.pallas import tpu as pltpu
from jax.experimental.pallas import tpu_sc as plsc

N_U = 5000
N_I = 5000
N_NODES = 10000
D = 128
DH = 64
E = 320000
NUM_LAYERS = 3

N_TILES = 16
RPT = N_NODES // N_TILES
ECH = 128
NCHUNK = E // ECH
CPT = NCHUNK // N_TILES
BLK = 13
NBLK = CPT // BLK
NGB = 4
NSEG = DH // 16

_RCH = [(o, min(80, RPT - o)) for o in range(0, RPT, 80)]


def _gcn_body(user_hbm, item_hbm, edge_hbm, out_hbm, xm_hbm,
              sh, deg_sh,
              xbuf, hbuf, gbuf, rbuf, sidx, didx, ones, zbuf,
              gsem, ssem, isem, dsem):
  c = lax.axis_index("c")
  s = lax.axis_index("s")
  col0 = c * DH
  r0 = s * RPT
  c0 = s * CPT
  xm = xm_hbm.at[c]

  zv = jnp.zeros((16,), jnp.float32)

  def _zbuf_fill(i, _):
    zbuf[i // NSEG, pl.ds((i % NSEG) * 16, 16)] = zv
    return 0
  lax.fori_loop(0, 80 * NSEG, _zbuf_fill, 0)

  def _ones_fill(i, _):
    ones[i, :] = zv + 1.0
    return 0
  lax.fori_loop(0, ECH, _ones_fill, 0)

  def zero_rows(dst_sh):
    for off, n in _RCH:
      pltpu.sync_copy(zbuf.at[pl.ds(0, n), :],
                      dst_sh.at[pl.ds(r0 + off, n), :])

  zero_rows(sh)
  for off, n in _RCH:
    pltpu.sync_copy(zbuf.at[pl.ds(0, n), pl.ds(0, 16)],
                    deg_sh.at[pl.ds(r0 + off, n), :])

  for off, n in _RCH:
    @pl.when(s < 8)
    def _():
      pltpu.sync_copy(user_hbm.at[pl.ds(r0 + off, n), pl.ds(col0, DH)],
                      xbuf.at[pl.ds(0, n), :])

    @pl.when(s >= 8)
    def _():
      pltpu.sync_copy(item_hbm.at[pl.ds(r0 - N_U + off, n), pl.ds(col0, DH)],
                      xbuf.at[pl.ds(0, n), :])

    pltpu.sync_copy(xbuf.at[pl.ds(0, n), :], xm.at[pl.ds(r0 + off, n), :])
    pltpu.sync_copy(xbuf.at[pl.ds(0, n), :],
                    out_hbm.at[pl.ds(r0 + off, n), pl.ds(col0, DH)])

  plsc.subcore_barrier()

  def edge_phase(count_deg):
    pltpu.sync_copy(edge_hbm.at[0, pl.ds(c0, BLK), :], sidx.at[0])
    pltpu.sync_copy(edge_hbm.at[1, pl.ds(c0, BLK), :], didx.at[0])

    def block(k, _):
      p = lax.rem(k, 2)
      pn = lax.rem(k + 1, 2)

      @pl.when(k + 1 < NBLK)
      def _():
        nb = c0 + BLK * (k + 1)
        pltpu.async_copy(edge_hbm.at[0, pl.ds(nb, BLK), :], sidx.at[pn], isem)
        pltpu.async_copy(edge_hbm.at[1, pl.ds(nb, BLK), :], didx.at[pn], isem)

      for j in range(min(NGB - 1, BLK)):
        pltpu.async_copy(xm.at[sidx.at[p, j]], gbuf.at[j % NGB], gsem)
      for j in range(BLK):
        g = j % NGB
        if j >= 1:
          pltpu.make_async_copy(gbuf.at[0], sh.at[didx.at[p, 0]],
                                ssem).wait()
        if j + NGB - 1 < BLK:
          pltpu.async_copy(xm.at[sidx.at[p, j + NGB - 1]],
                           gbuf.at[(j + NGB - 1) % NGB], gsem)
        pltpu.make_async_copy(xm.at[sidx.at[p, 0]], gbuf.at[g], gsem).wait()
        pltpu.async_copy(gbuf.at[g], sh.at[didx.at[p, j]], ssem, add=True)
        if count_deg:
          pltpu.async_copy(ones, deg_sh.at[didx.at[p, j]], dsem, add=True)

      pltpu.make_async_copy(gbuf.at[0], sh.at[didx.at[p, 0]], ssem).wait()
      if count_deg:
        for j in range(BLK):
          pltpu.make_async_copy(ones, deg_sh.at[didx.at[p, 0]], dsem).wait()

      @pl.when(k + 1 < NBLK)
      def _():
        pltpu.make_async_copy(edge_hbm.at[0, pl.ds(c0, BLK), :],
                              sidx.at[pn], isem).wait()
        pltpu.make_async_copy(edge_hbm.at[1, pl.ds(c0, BLK), :],
                              didx.at[pn], isem).wait()
      return 0

    lax.fori_loop(0, NBLK, block, 0)

    @pl.when(s < 4)
    def _():
      tb = N_TILES * CPT + s
      pltpu.sync_copy(edge_hbm.at[0, pl.ds(tb, 1), :], sidx.at[0, pl.ds(0, 1)])
      pltpu.sync_copy(edge_hbm.at[1, pl.ds(tb, 1), :], didx.at[0, pl.ds(0, 1)])
      pltpu.async_copy(xm.at[sidx.at[0, 0]], gbuf.at[0], gsem)
      pltpu.make_async_copy(xm.at[sidx.at[0, 0]], gbuf.at[0], gsem).wait()
      pltpu.sync_copy(gbuf.at[0], sh.at[didx.at[0, 0]], add=True)
      if count_deg:
        pltpu.sync_copy(ones, deg_sh.at[didx.at[0, 0]], add=True)

  def recip_phase():
    for off, n in _RCH:
      ro = r0 + off
      pltpu.sync_copy(deg_sh.at[pl.ds(ro, n), :], rbuf.at[pl.ds(0, n), :])
      def body(i, _):
        rbuf[i, :] = 1.0 / jnp.maximum(rbuf[i, :], 1.0)
        return 0
      lax.fori_loop(0, n, body, 0)
      pltpu.sync_copy(rbuf.at[pl.ds(0, n), :], deg_sh.at[pl.ds(ro, n), :])

  def norm_phase(last):
    for off, n in _RCH:
      ro = r0 + off
      pltpu.sync_copy(sh.at[pl.ds(ro, n), :], xbuf.at[pl.ds(0, n), :])
      pltpu.sync_copy(deg_sh.at[pl.ds(ro, n), :], rbuf.at[pl.ds(0, n), :])
      pltpu.sync_copy(out_hbm.at[pl.ds(ro, n), pl.ds(col0, DH)],
                      hbuf.at[pl.ds(0, n), :])
      def body(i, _):
        rec = rbuf[i, :]
        for j in range(NSEG):
          sl = pl.ds(j * 16, 16)
          x = xbuf[i, sl] * rec
          xbuf[i, sl] = x
          if last:
            hbuf[i, sl] = (hbuf[i, sl] + x) * 0.25
          else:
            hbuf[i, sl] = hbuf[i, sl] + x
        return 0
      lax.fori_loop(0, n, body, 0)
      if not last:
        pltpu.sync_copy(xbuf.at[pl.ds(0, n), :], xm.at[pl.ds(ro, n), :])
        pltpu.sync_copy(zbuf.at[pl.ds(0, n), :], sh.at[pl.ds(ro, n), :])
      pltpu.sync_copy(hbuf.at[pl.ds(0, n), :],
                      out_hbm.at[pl.ds(ro, n), pl.ds(col0, DH)])

  for layer in range(NUM_LAYERS):
    edge_phase(layer == 0)
    plsc.subcore_barrier()
    if layer == 0:
      recip_phase()
    norm_phase(layer == NUM_LAYERS - 1)
    if layer < NUM_LAYERS - 1:
      plsc.subcore_barrier()


@functools.partial(
    pl.kernel,
    out_type=(
        jax.ShapeDtypeStruct((N_NODES, D), jnp.float32),
        jax.ShapeDtypeStruct((2, N_NODES, DH), jnp.float32),
    ),
    mesh=plsc.VectorSubcoreMesh(core_axis_name="c", subcore_axis_name="s"),
    compiler_params=pltpu.CompilerParams(use_tc_tiling_on_sc=False),
    scratch_types=[
        pltpu.VMEM_SHARED((N_NODES, DH), jnp.float32),
        pltpu.VMEM_SHARED((N_NODES, 16), jnp.float32),
        pltpu.VMEM((80, DH), jnp.float32),
        pltpu.VMEM((80, DH), jnp.float32),
        pltpu.VMEM((NGB, ECH, DH), jnp.float32),
        pltpu.VMEM((80, 16), jnp.float32),
        pltpu.VMEM((2, BLK, ECH), jnp.int32),
        pltpu.VMEM((2, BLK, ECH), jnp.int32),
        pltpu.VMEM((ECH, 16), jnp.float32),
        pltpu.VMEM((80, DH), jnp.float32),
        pltpu.SemaphoreType.DMA,
        pltpu.SemaphoreType.DMA,
        pltpu.SemaphoreType.DMA,
        pltpu.SemaphoreType.DMA,
    ],
)
def _gcn(user_hbm, item_hbm, edge_hbm, out_hbm, xm_hbm, *scratch):
  _gcn_body(user_hbm, item_hbm, edge_hbm, out_hbm, xm_hbm, *scratch)


@jax.jit
def kernel(user_table, item_table, edge_index):
  out, _ = _gcn(user_table, item_table, edge_index.reshape(2, NCHUNK, ECH))
  return out[:N_U], out[N_U:]


# speedup vs baseline: 12.2693x; 1.0038x over previous
import functools

import jax
import jax.numpy as jnp
from jax import lax
from jax.experimental import pallas as pl
from jax.experimental.pallas import tpu as pltpu
from jax.experimental.pallas import tpu_sc as plsc

N_U = 5000
N_I = 5000
N_NODES = 10000
D = 128
DH = 64
E = 320000
NUM_LAYERS = 3

N_TILES = 16
RPT = N_NODES // N_TILES
ECH = 128
NCHUNK = E // ECH
CPT = NCHUNK // N_TILES
BLK = 13
NBLK = CPT // BLK
NGB = 6
NGD = 3
NSD = NGB - NGD
NSEG = DH // 16

_RCH = [(o, min(80, RPT - o)) for o in range(0, RPT, 80)]


def _gcn_body(user_hbm, item_hbm, edge_hbm, out_hbm, xm_hbm,
              sh, deg_sh,
              xbuf, hbuf, gbuf, rbuf, sidx, didx, ones, zbuf,
              gsem, ssem, isem, dsem):
  c = lax.axis_index("c")
  s = lax.axis_index("s")
  col0 = c * DH
  r0 = s * RPT
  c0 = s * CPT
  xm = xm_hbm.at[c]

  zv = jnp.zeros((16,), jnp.float32)

  def _zbuf_fill(i, _):
    zbuf[i // NSEG, pl.ds((i % NSEG) * 16, 16)] = zv
    return 0
  lax.fori_loop(0, 80 * NSEG, _zbuf_fill, 0)

  def _ones_fill(i, _):
    ones[i, :] = zv + 1.0
    return 0
  lax.fori_loop(0, ECH, _ones_fill, 0)

  def zero_rows(dst_sh):
    for off, n in _RCH:
      pltpu.sync_copy(zbuf.at[pl.ds(0, n), :],
                      dst_sh.at[pl.ds(r0 + off, n), :])

  zero_rows(sh)
  for off, n in _RCH:
    pltpu.sync_copy(zbuf.at[pl.ds(0, n), pl.ds(0, 16)],
                    deg_sh.at[pl.ds(r0 + off, n), :])

  for off, n in _RCH:
    @pl.when(s < 8)
    def _():
      pltpu.sync_copy(user_hbm.at[pl.ds(r0 + off, n), pl.ds(col0, DH)],
                      xbuf.at[pl.ds(0, n), :])

    @pl.when(s >= 8)
    def _():
      pltpu.sync_copy(item_hbm.at[pl.ds(r0 - N_U + off, n), pl.ds(col0, DH)],
                      xbuf.at[pl.ds(0, n), :])

    pltpu.sync_copy(xbuf.at[pl.ds(0, n), :], xm.at[pl.ds(r0 + off, n), :])
    pltpu.sync_copy(xbuf.at[pl.ds(0, n), :],
                    out_hbm.at[pl.ds(r0 + off, n), pl.ds(col0, DH)])

  plsc.subcore_barrier()

  def edge_phase(count_deg):
    pltpu.sync_copy(edge_hbm.at[0, pl.ds(c0, BLK), :], sidx.at[0])
    pltpu.sync_copy(edge_hbm.at[1, pl.ds(c0, BLK), :], didx.at[0])

    def block(k, _):
      p = lax.rem(k, 2)
      pn = lax.rem(k + 1, 2)

      @pl.when(k + 1 < NBLK)
      def _():
        nb = c0 + BLK * (k + 1)
        pltpu.async_copy(edge_hbm.at[0, pl.ds(nb, BLK), :], sidx.at[pn], isem)
        pltpu.async_copy(edge_hbm.at[1, pl.ds(nb, BLK), :], didx.at[pn], isem)

      for j in range(min(NGD, BLK)):
        pltpu.async_copy(xm.at[sidx.at[p, j]], gbuf.at[j % NGB], gsem)
      for j in range(BLK):
        g = j % NGB
        if j >= NSD:
          pltpu.make_async_copy(gbuf.at[0], sh.at[didx.at[p, 0]],
                                ssem).wait()
        if j + NGD < BLK:
          pltpu.async_copy(xm.at[sidx.at[p, j + NGD]],
                           gbuf.at[(j + NGD) % NGB], gsem)
        pltpu.make_async_copy(xm.at[sidx.at[p, 0]], gbuf.at[g], gsem).wait()
        pltpu.async_copy(gbuf.at[g], sh.at[didx.at[p, j]], ssem, add=True)
        if count_deg:
          pltpu.async_copy(ones, deg_sh.at[didx.at[p, j]], dsem, add=True)

      for j in range(min(NSD, BLK)):
        pltpu.make_async_copy(gbuf.at[0], sh.at[didx.at[p, 0]], ssem).wait()
      if count_deg:
        for j in range(BLK):
          pltpu.make_async_copy(ones, deg_sh.at[didx.at[p, 0]], dsem).wait()

      @pl.when(k + 1 < NBLK)
      def _():
        pltpu.make_async_copy(edge_hbm.at[0, pl.ds(c0, BLK), :],
                              sidx.at[pn], isem).wait()
        pltpu.make_async_copy(edge_hbm.at[1, pl.ds(c0, BLK), :],
                              didx.at[pn], isem).wait()
      return 0

    lax.fori_loop(0, NBLK, block, 0)

    @pl.when(s < 4)
    def _():
      tb = N_TILES * CPT + s
      pltpu.sync_copy(edge_hbm.at[0, pl.ds(tb, 1), :], sidx.at[0, pl.ds(0, 1)])
      pltpu.sync_copy(edge_hbm.at[1, pl.ds(tb, 1), :], didx.at[0, pl.ds(0, 1)])
      pltpu.async_copy(xm.at[sidx.at[0, 0]], gbuf.at[0], gsem)
      pltpu.make_async_copy(xm.at[sidx.at[0, 0]], gbuf.at[0], gsem).wait()
      pltpu.sync_copy(gbuf.at[0], sh.at[didx.at[0, 0]], add=True)
      if count_deg:
        pltpu.sync_copy(ones, deg_sh.at[didx.at[0, 0]], add=True)

  def recip_phase():
    for off, n in _RCH:
      ro = r0 + off
      pltpu.sync_copy(deg_sh.at[pl.ds(ro, n), :], rbuf.at[pl.ds(0, n), :])
      def body(i, _):
        rbuf[i, :] = 1.0 / jnp.maximum(rbuf[i, :], 1.0)
        return 0
      lax.fori_loop(0, n, body, 0)
      pltpu.sync_copy(rbuf.at[pl.ds(0, n), :], deg_sh.at[pl.ds(ro, n), :])

  def norm_phase(last):
    for off, n in _RCH:
      ro = r0 + off
      pltpu.sync_copy(sh.at[pl.ds(ro, n), :], xbuf.at[pl.ds(0, n), :])
      pltpu.sync_copy(deg_sh.at[pl.ds(ro, n), :], rbuf.at[pl.ds(0, n), :])
      pltpu.sync_copy(out_hbm.at[pl.ds(ro, n), pl.ds(col0, DH)],
                      hbuf.at[pl.ds(0, n), :])
      def body(i, _):
        rec = rbuf[i, :]
        for j in range(NSEG):
          sl = pl.ds(j * 16, 16)
          x = xbuf[i, sl] * rec
          xbuf[i, sl] = x
          if last:
            hbuf[i, sl] = (hbuf[i, sl] + x) * 0.25
          else:
            hbuf[i, sl] = hbuf[i, sl] + x
        return 0
      lax.fori_loop(0, n, body, 0)
      if not last:
        pltpu.sync_copy(xbuf.at[pl.ds(0, n), :], xm.at[pl.ds(ro, n), :])
        pltpu.sync_copy(zbuf.at[pl.ds(0, n), :], sh.at[pl.ds(ro, n), :])
      pltpu.sync_copy(hbuf.at[pl.ds(0, n), :],
                      out_hbm.at[pl.ds(ro, n), pl.ds(col0, DH)])

  for layer in range(NUM_LAYERS):
    edge_phase(layer == 0)
    plsc.subcore_barrier()
    if layer == 0:
      recip_phase()
    norm_phase(layer == NUM_LAYERS - 1)
    if layer < NUM_LAYERS - 1:
      plsc.subcore_barrier()


@functools.partial(
    pl.kernel,
    out_type=(
        jax.ShapeDtypeStruct((N_NODES, D), jnp.float32),
        jax.ShapeDtypeStruct((2, N_NODES, DH), jnp.float32),
    ),
    mesh=plsc.VectorSubcoreMesh(core_axis_name="c", subcore_axis_name="s"),
    compiler_params=pltpu.CompilerParams(use_tc_tiling_on_sc=False),
    scratch_types=[
        pltpu.VMEM_SHARED((N_NODES, DH), jnp.float32),
        pltpu.VMEM_SHARED((N_NODES, 16), jnp.float32),
        pltpu.VMEM((80, DH), jnp.float32),
        pltpu.VMEM((80, DH), jnp.float32),
        pltpu.VMEM((NGB, ECH, DH), jnp.float32),
        pltpu.VMEM((80, 16), jnp.float32),
        pltpu.VMEM((2, BLK, ECH), jnp.int32),
        pltpu.VMEM((2, BLK, ECH), jnp.int32),
        pltpu.VMEM((ECH, 16), jnp.float32),
        pltpu.VMEM((80, DH), jnp.float32),
        pltpu.SemaphoreType.DMA,
        pltpu.SemaphoreType.DMA,
        pltpu.SemaphoreType.DMA,
        pltpu.SemaphoreType.DMA,
    ],
)
def _gcn(user_hbm, item_hbm, edge_hbm, out_hbm, xm_hbm, *scratch):
  _gcn_body(user_hbm, item_hbm, edge_hbm, out_hbm, xm_hbm, *scratch)


@jax.jit
def kernel(user_table, item_table, edge_index):
  out, _ = _gcn(user_table, item_table, edge_index.reshape(2, NCHUNK, ECH))
  return out[:N_U], out[N_U:]


# speedup vs baseline: 12.7099x; 1.0359x over previous
import functools

import jax
import jax.numpy as jnp
from jax import lax
from jax.experimental import pallas as pl
from jax.experimental.pallas import tpu as pltpu
from jax.experimental.pallas import tpu_sc as plsc

N_U = 5000
N_I = 5000
N_NODES = 10000
D = 128
DH = 64
E = 320000
NUM_LAYERS = 3

N_TILES = 16
RPT = N_NODES // N_TILES
ECH = 128
NCHUNK = E // ECH
CPT = NCHUNK // N_TILES
BLK = 26
NBLK = CPT // BLK
NGB = 5
NGD = 3
NSD = NGB - NGD
NSEG = DH // 16

_RCH = [(o, min(80, RPT - o)) for o in range(0, RPT, 80)]


def _gcn_body(user_hbm, item_hbm, edge_hbm, out_hbm, xm_hbm,
              sh, deg_sh,
              xbuf, hbuf, gbuf, rbuf, sidx, didx, ones, zbuf,
              gsem, ssem, isem, dsem):
  c = lax.axis_index("c")
  s = lax.axis_index("s")
  col0 = c * DH
  r0 = s * RPT
  c0 = s * CPT
  xm = xm_hbm.at[c]

  zv = jnp.zeros((16,), jnp.float32)

  def _zbuf_fill(i, _):
    zbuf[i // NSEG, pl.ds((i % NSEG) * 16, 16)] = zv
    return 0
  lax.fori_loop(0, 80 * NSEG, _zbuf_fill, 0)

  def _ones_fill(i, _):
    ones[i, :] = zv + 1.0
    return 0
  lax.fori_loop(0, ECH, _ones_fill, 0)

  def zero_rows(dst_sh):
    for off, n in _RCH:
      pltpu.sync_copy(zbuf.at[pl.ds(0, n), :],
                      dst_sh.at[pl.ds(r0 + off, n), :])

  zero_rows(sh)
  for off, n in _RCH:
    pltpu.sync_copy(zbuf.at[pl.ds(0, n), pl.ds(0, 16)],
                    deg_sh.at[pl.ds(r0 + off, n), :])

  for off, n in _RCH:
    @pl.when(s < 8)
    def _():
      pltpu.sync_copy(user_hbm.at[pl.ds(r0 + off, n), pl.ds(col0, DH)],
                      xbuf.at[pl.ds(0, n), :])

    @pl.when(s >= 8)
    def _():
      pltpu.sync_copy(item_hbm.at[pl.ds(r0 - N_U + off, n), pl.ds(col0, DH)],
                      xbuf.at[pl.ds(0, n), :])

    pltpu.sync_copy(xbuf.at[pl.ds(0, n), :], xm.at[pl.ds(r0 + off, n), :])
    pltpu.sync_copy(xbuf.at[pl.ds(0, n), :],
                    out_hbm.at[pl.ds(r0 + off, n), pl.ds(col0, DH)])

  plsc.subcore_barrier()

  def edge_phase(count_deg):
    pltpu.sync_copy(edge_hbm.at[0, pl.ds(c0, BLK), :], sidx.at[0])
    pltpu.sync_copy(edge_hbm.at[1, pl.ds(c0, BLK), :], didx.at[0])

    def block(k, _):
      p = lax.rem(k, 2)
      pn = lax.rem(k + 1, 2)

      @pl.when(k + 1 < NBLK)
      def _():
        nb = c0 + BLK * (k + 1)
        pltpu.async_copy(edge_hbm.at[0, pl.ds(nb, BLK), :], sidx.at[pn], isem)
        pltpu.async_copy(edge_hbm.at[1, pl.ds(nb, BLK), :], didx.at[pn], isem)

      for j in range(min(NGD, BLK)):
        pltpu.async_copy(xm.at[sidx.at[p, j]], gbuf.at[j % NGB], gsem)
      for j in range(BLK):
        g = j % NGB
        if j >= NSD:
          pltpu.make_async_copy(gbuf.at[0], sh.at[didx.at[p, 0]],
                                ssem).wait()
        if j + NGD < BLK:
          pltpu.async_copy(xm.at[sidx.at[p, j + NGD]],
                           gbuf.at[(j + NGD) % NGB], gsem)
        pltpu.make_async_copy(xm.at[sidx.at[p, 0]], gbuf.at[g], gsem).wait()
        pltpu.async_copy(gbuf.at[g], sh.at[didx.at[p, j]], ssem, add=True)
        if count_deg:
          pltpu.async_copy(ones, deg_sh.at[didx.at[p, j]], dsem, add=True)

      for j in range(min(NSD, BLK)):
        pltpu.make_async_copy(gbuf.at[0], sh.at[didx.at[p, 0]], ssem).wait()
      if count_deg:
        for j in range(BLK):
          pltpu.make_async_copy(ones, deg_sh.at[didx.at[p, 0]], dsem).wait()

      @pl.when(k + 1 < NBLK)
      def _():
        pltpu.make_async_copy(edge_hbm.at[0, pl.ds(c0, BLK), :],
                              sidx.at[pn], isem).wait()
        pltpu.make_async_copy(edge_hbm.at[1, pl.ds(c0, BLK), :],
                              didx.at[pn], isem).wait()
      return 0

    lax.fori_loop(0, NBLK, block, 0)

    @pl.when(s < 4)
    def _():
      tb = N_TILES * CPT + s
      pltpu.sync_copy(edge_hbm.at[0, pl.ds(tb, 1), :], sidx.at[0, pl.ds(0, 1)])
      pltpu.sync_copy(edge_hbm.at[1, pl.ds(tb, 1), :], didx.at[0, pl.ds(0, 1)])
      pltpu.async_copy(xm.at[sidx.at[0, 0]], gbuf.at[0], gsem)
      pltpu.make_async_copy(xm.at[sidx.at[0, 0]], gbuf.at[0], gsem).wait()
      pltpu.sync_copy(gbuf.at[0], sh.at[didx.at[0, 0]], add=True)
      if count_deg:
        pltpu.sync_copy(ones, deg_sh.at[didx.at[0, 0]], add=True)

  def norm_phase(first, last):
    for off, n in _RCH:
      ro = r0 + off
      pltpu.sync_copy(sh.at[pl.ds(ro, n), :], xbuf.at[pl.ds(0, n), :])
      pltpu.sync_copy(deg_sh.at[pl.ds(ro, n), :], rbuf.at[pl.ds(0, n), :])
      pltpu.sync_copy(out_hbm.at[pl.ds(ro, n), pl.ds(col0, DH)],
                      hbuf.at[pl.ds(0, n), :])
      if first:
        def rbody(i, _):
          rbuf[i, :] = 1.0 / jnp.maximum(rbuf[i, :], 1.0)
          return 0
        lax.fori_loop(0, n, rbody, 0)
      def body(i, _):
        rec = rbuf[i, :]
        for j in range(NSEG):
          sl = pl.ds(j * 16, 16)
          x = xbuf[i, sl] * rec
          xbuf[i, sl] = x
          if last:
            hbuf[i, sl] = (hbuf[i, sl] + x) * 0.25
          else:
            hbuf[i, sl] = hbuf[i, sl] + x
        return 0
      lax.fori_loop(0, n, body, 0)
      if first:
        pltpu.sync_copy(rbuf.at[pl.ds(0, n), :], deg_sh.at[pl.ds(ro, n), :])
      if not last:
        pltpu.sync_copy(xbuf.at[pl.ds(0, n), :], xm.at[pl.ds(ro, n), :])
        pltpu.sync_copy(zbuf.at[pl.ds(0, n), :], sh.at[pl.ds(ro, n), :])
      pltpu.sync_copy(hbuf.at[pl.ds(0, n), :],
                      out_hbm.at[pl.ds(ro, n), pl.ds(col0, DH)])

  for layer in range(NUM_LAYERS):
    edge_phase(layer == 0)
    plsc.subcore_barrier()
    norm_phase(layer == 0, layer == NUM_LAYERS - 1)
    if layer < NUM_LAYERS - 1:
      plsc.subcore_barrier()


@functools.partial(
    pl.kernel,
    out_type=(
        jax.ShapeDtypeStruct((N_NODES, D), jnp.float32),
        jax.ShapeDtypeStruct((2, N_NODES, DH), jnp.float32),
    ),
    mesh=plsc.VectorSubcoreMesh(core_axis_name="c", subcore_axis_name="s"),
    compiler_params=pltpu.CompilerParams(use_tc_tiling_on_sc=False),
    scratch_types=[
        pltpu.VMEM_SHARED((N_NODES, DH), jnp.float32),
        pltpu.VMEM_SHARED((N_NODES, 16), jnp.float32),
        pltpu.VMEM((80, DH), jnp.float32),
        pltpu.VMEM((80, DH), jnp.float32),
        pltpu.VMEM((NGB, ECH, DH), jnp.float32),
        pltpu.VMEM((80, 16), jnp.float32),
        pltpu.VMEM((2, BLK, ECH), jnp.int32),
        pltpu.VMEM((2, BLK, ECH), jnp.int32),
        pltpu.VMEM((ECH, 16), jnp.float32),
        pltpu.VMEM((80, DH), jnp.float32),
        pltpu.SemaphoreType.DMA,
        pltpu.SemaphoreType.DMA,
        pltpu.SemaphoreType.DMA,
        pltpu.SemaphoreType.DMA,
    ],
)
def _gcn(user_hbm, item_hbm, edge_hbm, out_hbm, xm_hbm, *scratch):
  _gcn_body(user_hbm, item_hbm, edge_hbm, out_hbm, xm_hbm, *scratch)


@jax.jit
def kernel(user_table, item_table, edge_index):
  out, _ = _gcn(user_table, item_table, edge_index.reshape(2, NCHUNK, ECH))
  return out[:N_U], out[N_U:]


# speedup vs baseline: 12.9316x; 1.0174x over previous
import functools

import jax
import jax.numpy as jnp
from jax import lax
from jax.experimental import pallas as pl
from jax.experimental.pallas import tpu as pltpu
from jax.experimental.pallas import tpu_sc as plsc

N_U = 5000
N_I = 5000
N_NODES = 10000
D = 128
DH = 64
E = 320000
NUM_LAYERS = 3

N_TILES = 16
RPT = N_NODES // N_TILES
ECH = 128
NCHUNK = E // ECH
CPT = NCHUNK // N_TILES
BLK = 26
NBLK = CPT // BLK
NGB = 5
NGD = 3
NSD = NGB - NGD
NSEG = DH // 16

_RCH = [(o, min(80, RPT - o)) for o in range(0, RPT, 80)]


def _gcn_body(user_hbm, item_hbm, edge_hbm, out_hbm, xm_hbm,
              sh, deg_sh,
              xbuf, hbuf, gbuf, rbuf, sidx, didx, ones, zbuf,
              gsem, ssem, isem, dsem):
  c = lax.axis_index("c")
  s = lax.axis_index("s")
  col0 = c * DH
  r0 = s * RPT
  c0 = s * CPT
  xm = xm_hbm.at[c]

  zv = jnp.zeros((16,), jnp.float32)

  def _zbuf_fill(i, _):
    zbuf[i // NSEG, pl.ds((i % NSEG) * 16, 16)] = zv
    return 0
  lax.fori_loop(0, 80 * NSEG, _zbuf_fill, 0)

  def _ones_fill(i, _):
    ones[i, :] = zv + 1.0
    return 0
  lax.fori_loop(0, ECH, _ones_fill, 0)

  def zero_rows(dst_sh):
    for off, n in _RCH:
      pltpu.sync_copy(zbuf.at[pl.ds(0, n), :],
                      dst_sh.at[pl.ds(r0 + off, n), :])

  zero_rows(sh)
  for off, n in _RCH:
    pltpu.sync_copy(zbuf.at[pl.ds(0, n), pl.ds(0, 16)],
                    deg_sh.at[pl.ds(r0 + off, n), :])

  for off, n in _RCH:
    @pl.when(s < 8)
    def _():
      pltpu.sync_copy(user_hbm.at[pl.ds(r0 + off, n), pl.ds(col0, DH)],
                      xbuf.at[pl.ds(0, n), :])

    @pl.when(s >= 8)
    def _():
      pltpu.sync_copy(item_hbm.at[pl.ds(r0 - N_U + off, n), pl.ds(col0, DH)],
                      xbuf.at[pl.ds(0, n), :])

    pltpu.sync_copy(xbuf.at[pl.ds(0, n), :], xm.at[pl.ds(r0 + off, n), :])
    pltpu.sync_copy(xbuf.at[pl.ds(0, n), :],
                    out_hbm.at[pl.ds(r0 + off, n), pl.ds(col0, DH)])

  plsc.subcore_barrier()

  def edge_phase(count_deg):
    pltpu.sync_copy(edge_hbm.at[0, pl.ds(c0, BLK), :], sidx.at[0])
    pltpu.sync_copy(edge_hbm.at[1, pl.ds(c0, BLK), :], didx.at[0])

    def block(k, _):
      p = lax.rem(k, 2)
      pn = lax.rem(k + 1, 2)

      @pl.when(k + 1 < NBLK)
      def _():
        nb = c0 + BLK * (k + 1)
        pltpu.async_copy(edge_hbm.at[0, pl.ds(nb, BLK), :], sidx.at[pn], isem)
        pltpu.async_copy(edge_hbm.at[1, pl.ds(nb, BLK), :], didx.at[pn], isem)

      for j in range(min(NGD, BLK)):
        pltpu.async_copy(xm.at[sidx.at[p, j]], gbuf.at[j % NGB], gsem)
      for j in range(BLK):
        g = j % NGB
        if j >= NSD:
          pltpu.make_async_copy(gbuf.at[0], sh.at[didx.at[p, 0]],
                                ssem).wait()
        if j + NGD < BLK:
          pltpu.async_copy(xm.at[sidx.at[p, j + NGD]],
                           gbuf.at[(j + NGD) % NGB], gsem)
        pltpu.make_async_copy(xm.at[sidx.at[p, 0]], gbuf.at[g], gsem).wait()
        pltpu.async_copy(gbuf.at[g], sh.at[didx.at[p, j]], ssem, add=True)
        if count_deg:
          pltpu.async_copy(ones, deg_sh.at[didx.at[p, j]], dsem, add=True)

      for j in range(min(NSD, BLK)):
        pltpu.make_async_copy(gbuf.at[0], sh.at[didx.at[p, 0]], ssem).wait()
      if count_deg:
        for j in range(BLK):
          pltpu.make_async_copy(ones, deg_sh.at[didx.at[p, 0]], dsem).wait()

      @pl.when(k + 1 < NBLK)
      def _():
        pltpu.make_async_copy(edge_hbm.at[0, pl.ds(c0, BLK), :],
                              sidx.at[pn], isem).wait()
        pltpu.make_async_copy(edge_hbm.at[1, pl.ds(c0, BLK), :],
                              didx.at[pn], isem).wait()
      return 0

    lax.fori_loop(0, NBLK, block, 0)

    @pl.when(s < 4)
    def _():
      tb = N_TILES * CPT + s
      pltpu.sync_copy(edge_hbm.at[0, pl.ds(tb, 1), :], sidx.at[0, pl.ds(0, 1)])
      pltpu.sync_copy(edge_hbm.at[1, pl.ds(tb, 1), :], didx.at[0, pl.ds(0, 1)])
      pltpu.async_copy(xm.at[sidx.at[0, 0]], gbuf.at[0], gsem)
      pltpu.make_async_copy(xm.at[sidx.at[0, 0]], gbuf.at[0], gsem).wait()
      pltpu.sync_copy(gbuf.at[0], sh.at[didx.at[0, 0]], add=True)
      if count_deg:
        pltpu.sync_copy(ones, deg_sh.at[didx.at[0, 0]], add=True)

  def norm_phase(first, last):
    for off, n in _RCH:
      ro = r0 + off
      pltpu.sync_copy(sh.at[pl.ds(ro, n), :], xbuf.at[pl.ds(0, n), :])
      pltpu.sync_copy(deg_sh.at[pl.ds(ro, n), :], rbuf.at[pl.ds(0, n), :])
      pltpu.sync_copy(out_hbm.at[pl.ds(ro, n), pl.ds(col0, DH)],
                      hbuf.at[pl.ds(0, n), :])
      if first:
        @plsc.parallel_loop(0, n, unroll=4)
        def _(i):
          rbuf[i, :] = 1.0 / jnp.maximum(rbuf[i, :], 1.0)

      @plsc.parallel_loop(0, n, unroll=4)
      def _(i):
        rec = rbuf[i, :]
        for j in range(NSEG):
          sl = pl.ds(j * 16, 16)
          x = xbuf[i, sl] * rec
          xbuf[i, sl] = x
          if last:
            hbuf[i, sl] = (hbuf[i, sl] + x) * 0.25
          else:
            hbuf[i, sl] = hbuf[i, sl] + x
      if first:
        pltpu.sync_copy(rbuf.at[pl.ds(0, n), :], deg_sh.at[pl.ds(ro, n), :])
      if not last:
        pltpu.sync_copy(xbuf.at[pl.ds(0, n), :], xm.at[pl.ds(ro, n), :])
        pltpu.sync_copy(zbuf.at[pl.ds(0, n), :], sh.at[pl.ds(ro, n), :])
      pltpu.sync_copy(hbuf.at[pl.ds(0, n), :],
                      out_hbm.at[pl.ds(ro, n), pl.ds(col0, DH)])

  for layer in range(NUM_LAYERS):
    edge_phase(layer == 0)
    plsc.subcore_barrier()
    norm_phase(layer == 0, layer == NUM_LAYERS - 1)
    if layer < NUM_LAYERS - 1:
      plsc.subcore_barrier()


@functools.partial(
    pl.kernel,
    out_type=(
        jax.ShapeDtypeStruct((N_NODES, D), jnp.float32),
        jax.ShapeDtypeStruct((2, N_NODES, DH), jnp.float32),
    ),
    mesh=plsc.VectorSubcoreMesh(core_axis_name="c", subcore_axis_name="s"),
    compiler_params=pltpu.CompilerParams(use_tc_tiling_on_sc=False),
    scratch_types=[
        pltpu.VMEM_SHARED((N_NODES, DH), jnp.float32),
        pltpu.VMEM_SHARED((N_NODES, 16), jnp.float32),
        pltpu.VMEM((80, DH), jnp.float32),
        pltpu.VMEM((80, DH), jnp.float32),
        pltpu.VMEM((NGB, ECH, DH), jnp.float32),
        pltpu.VMEM((80, 16), jnp.float32),
        pltpu.VMEM((2, BLK, ECH), jnp.int32),
        pltpu.VMEM((2, BLK, ECH), jnp.int32),
        pltpu.VMEM((ECH, 16), jnp.float32),
        pltpu.VMEM((80, DH), jnp.float32),
        pltpu.SemaphoreType.DMA,
        pltpu.SemaphoreType.DMA,
        pltpu.SemaphoreType.DMA,
        pltpu.SemaphoreType.DMA,
    ],
)
def _gcn(user_hbm, item_hbm, edge_hbm, out_hbm, xm_hbm, *scratch):
  _gcn_body(user_hbm, item_hbm, edge_hbm, out_hbm, xm_hbm, *scratch)


@jax.jit
def kernel(user_table, item_table, edge_index):
  out, _ = _gcn(user_table, item_table, edge_index.reshape(2, NCHUNK, ECH))
  return out[:N_U], out[N_U:]


# speedup vs baseline: 12.9452x; 1.0010x over previous
import functools

import jax
import jax.numpy as jnp
from jax import lax
from jax.experimental import pallas as pl
from jax.experimental.pallas import tpu as pltpu
from jax.experimental.pallas import tpu_sc as plsc

N_U = 5000
N_I = 5000
N_NODES = 10000
D = 128
DH = 64
E = 320000
NUM_LAYERS = 3

N_TILES = 16
RPT = N_NODES // N_TILES
ECH = 128
NCHUNK = E // ECH
CPT = NCHUNK // N_TILES
BLK = 26
NBLK = CPT // BLK
NGB = 5
NGD = 3
NSD = NGB - NGD
NSEG = DH // 16

_RCH = [(o, min(80, RPT - o)) for o in range(0, RPT, 80)]


def _gcn_body(user_hbm, item_hbm, edge_hbm, out_hbm, xm_hbm,
              sh, deg_sh,
              xbuf, hbuf, gbuf, rbuf, sidx, didx, ones, zbuf,
              gsem, ssem, isem, dsem):
  c = lax.axis_index("c")
  s = lax.axis_index("s")
  col0 = c * DH
  r0 = s * RPT
  c0 = s * CPT
  xm = xm_hbm.at[c]

  zv = jnp.zeros((16,), jnp.float32)

  def _zbuf_fill(i, _):
    zbuf[i // NSEG, pl.ds((i % NSEG) * 16, 16)] = zv
    return 0
  lax.fori_loop(0, 80 * NSEG, _zbuf_fill, 0)

  def _ones_fill(i, _):
    ones[i, :] = zv + 1.0
    return 0
  lax.fori_loop(0, ECH, _ones_fill, 0)

  def zero_rows(dst_sh):
    for off, n in _RCH:
      pltpu.sync_copy(zbuf.at[pl.ds(0, n), :],
                      dst_sh.at[pl.ds(r0 + off, n), :])

  setup_scope = jax.named_scope("setup")
  setup_scope.__enter__()
  zero_rows(sh)
  for off, n in _RCH:
    pltpu.sync_copy(zbuf.at[pl.ds(0, n), pl.ds(0, 16)],
                    deg_sh.at[pl.ds(r0 + off, n), :])

  for off, n in _RCH:
    @pl.when(s < 8)
    def _():
      pltpu.sync_copy(user_hbm.at[pl.ds(r0 + off, n), pl.ds(col0, DH)],
                      xbuf.at[pl.ds(0, n), :])

    @pl.when(s >= 8)
    def _():
      pltpu.sync_copy(item_hbm.at[pl.ds(r0 - N_U + off, n), pl.ds(col0, DH)],
                      xbuf.at[pl.ds(0, n), :])

    pltpu.sync_copy(xbuf.at[pl.ds(0, n), :], xm.at[pl.ds(r0 + off, n), :])
    pltpu.sync_copy(xbuf.at[pl.ds(0, n), :],
                    out_hbm.at[pl.ds(r0 + off, n), pl.ds(col0, DH)])

  plsc.subcore_barrier()
  setup_scope.__exit__(None, None, None)

  def edge_phase(count_deg):
    pltpu.sync_copy(edge_hbm.at[0, pl.ds(c0, BLK), :], sidx.at[0])
    pltpu.sync_copy(edge_hbm.at[1, pl.ds(c0, BLK), :], didx.at[0])

    def block(k, _):
      p = lax.rem(k, 2)
      pn = lax.rem(k + 1, 2)

      @pl.when(k + 1 < NBLK)
      def _():
        nb = c0 + BLK * (k + 1)
        pltpu.async_copy(edge_hbm.at[0, pl.ds(nb, BLK), :], sidx.at[pn], isem)
        pltpu.async_copy(edge_hbm.at[1, pl.ds(nb, BLK), :], didx.at[pn], isem)

      for j in range(min(NGD, BLK)):
        pltpu.async_copy(xm.at[sidx.at[p, j]], gbuf.at[j % NGB], gsem)
      for j in range(BLK):
        g = j % NGB
        if j >= NSD:
          pltpu.make_async_copy(gbuf.at[0], sh.at[didx.at[p, 0]],
                                ssem).wait()
        if j + NGD < BLK:
          pltpu.async_copy(xm.at[sidx.at[p, j + NGD]],
                           gbuf.at[(j + NGD) % NGB], gsem)
        pltpu.make_async_copy(xm.at[sidx.at[p, 0]], gbuf.at[g], gsem).wait()
        pltpu.async_copy(gbuf.at[g], sh.at[didx.at[p, j]], ssem, add=True)
        if count_deg:
          pltpu.async_copy(ones, deg_sh.at[didx.at[p, j]], dsem, add=True)

      for j in range(min(NSD, BLK)):
        pltpu.make_async_copy(gbuf.at[0], sh.at[didx.at[p, 0]], ssem).wait()
      if count_deg:
        for j in range(BLK):
          pltpu.make_async_copy(ones, deg_sh.at[didx.at[p, 0]], dsem).wait()

      @pl.when(k + 1 < NBLK)
      def _():
        pltpu.make_async_copy(edge_hbm.at[0, pl.ds(c0, BLK), :],
                              sidx.at[pn], isem).wait()
        pltpu.make_async_copy(edge_hbm.at[1, pl.ds(c0, BLK), :],
                              didx.at[pn], isem).wait()
      return 0

    lax.fori_loop(0, NBLK, block, 0)

    @pl.when(s < 4)
    def _():
      tb = N_TILES * CPT + s
      pltpu.sync_copy(edge_hbm.at[0, pl.ds(tb, 1), :], sidx.at[0, pl.ds(0, 1)])
      pltpu.sync_copy(edge_hbm.at[1, pl.ds(tb, 1), :], didx.at[0, pl.ds(0, 1)])
      pltpu.async_copy(xm.at[sidx.at[0, 0]], gbuf.at[0], gsem)
      pltpu.make_async_copy(xm.at[sidx.at[0, 0]], gbuf.at[0], gsem).wait()
      pltpu.sync_copy(gbuf.at[0], sh.at[didx.at[0, 0]], add=True)
      if count_deg:
        pltpu.sync_copy(ones, deg_sh.at[didx.at[0, 0]], add=True)

  def norm_phase(first, last):
    for off, n in _RCH:
      ro = r0 + off
      pltpu.sync_copy(sh.at[pl.ds(ro, n), :], xbuf.at[pl.ds(0, n), :])
      pltpu.sync_copy(deg_sh.at[pl.ds(ro, n), :], rbuf.at[pl.ds(0, n), :])
      pltpu.sync_copy(out_hbm.at[pl.ds(ro, n), pl.ds(col0, DH)],
                      hbuf.at[pl.ds(0, n), :])
      if first:
        @plsc.parallel_loop(0, n, unroll=4)
        def _(i):
          rbuf[i, :] = 1.0 / jnp.maximum(rbuf[i, :], 1.0)

      @plsc.parallel_loop(0, n, unroll=4)
      def _(i):
        rec = rbuf[i, :]
        for j in range(NSEG):
          sl = pl.ds(j * 16, 16)
          x = xbuf[i, sl] * rec
          xbuf[i, sl] = x
          if last:
            hbuf[i, sl] = (hbuf[i, sl] + x) * 0.25
          else:
            hbuf[i, sl] = hbuf[i, sl] + x
      if first:
        pltpu.sync_copy(rbuf.at[pl.ds(0, n), :], deg_sh.at[pl.ds(ro, n), :])
      if not last:
        pltpu.sync_copy(xbuf.at[pl.ds(0, n), :], xm.at[pl.ds(ro, n), :])
        pltpu.sync_copy(zbuf.at[pl.ds(0, n), :], sh.at[pl.ds(ro, n), :])
      pltpu.sync_copy(hbuf.at[pl.ds(0, n), :],
                      out_hbm.at[pl.ds(ro, n), pl.ds(col0, DH)])

  for layer in range(NUM_LAYERS):
    with jax.named_scope(f"edge{layer}"):
      edge_phase(layer == 0)
      plsc.subcore_barrier()
    with jax.named_scope(f"norm{layer}"):
      norm_phase(layer == 0, layer == NUM_LAYERS - 1)
      if layer < NUM_LAYERS - 1:
        plsc.subcore_barrier()


@functools.partial(
    pl.kernel,
    out_type=(
        jax.ShapeDtypeStruct((N_NODES, D), jnp.float32),
        jax.ShapeDtypeStruct((2, N_NODES, DH), jnp.float32),
    ),
    mesh=plsc.VectorSubcoreMesh(core_axis_name="c", subcore_axis_name="s"),
    compiler_params=pltpu.CompilerParams(use_tc_tiling_on_sc=False),
    scratch_types=[
        pltpu.VMEM_SHARED((N_NODES, DH), jnp.float32),
        pltpu.VMEM_SHARED((N_NODES, 16), jnp.float32),
        pltpu.VMEM((80, DH), jnp.float32),
        pltpu.VMEM((80, DH), jnp.float32),
        pltpu.VMEM((NGB, ECH, DH), jnp.float32),
        pltpu.VMEM((80, 16), jnp.float32),
        pltpu.VMEM((2, BLK, ECH), jnp.int32),
        pltpu.VMEM((2, BLK, ECH), jnp.int32),
        pltpu.VMEM((ECH, 16), jnp.float32),
        pltpu.VMEM((80, DH), jnp.float32),
        pltpu.SemaphoreType.DMA,
        pltpu.SemaphoreType.DMA,
        pltpu.SemaphoreType.DMA,
        pltpu.SemaphoreType.DMA,
    ],
)
def _gcn(user_hbm, item_hbm, edge_hbm, out_hbm, xm_hbm, *scratch):
  _gcn_body(user_hbm, item_hbm, edge_hbm, out_hbm, xm_hbm, *scratch)


@jax.jit
def kernel(user_table, item_table, edge_index):
  out, _ = _gcn(user_table, item_table, edge_index.reshape(2, NCHUNK, ECH))
  return out[:N_U], out[N_U:]


# speedup vs baseline: 13.3065x; 1.0279x over previous
import functools

import jax
import jax.numpy as jnp
from jax import lax
from jax.experimental import pallas as pl
from jax.experimental.pallas import tpu as pltpu
from jax.experimental.pallas import tpu_sc as plsc

N_U = 5000
N_I = 5000
N_NODES = 10000
D = 128
DH = 64
E = 320000
NUM_LAYERS = 3

N_TILES = 16
RPT = N_NODES // N_TILES
ECH = 128
NCHUNK = E // ECH
CPT = NCHUNK // N_TILES
BLK = 26
NBLK = CPT // BLK
NGB = 5
NGD = 3
NSD = NGB - NGD
NSEG = DH // 16

_RCH = [(o, min(80, RPT - o)) for o in range(0, RPT, 80)]


def _gcn_body(user_hbm, item_hbm, edge_hbm, uout, iout, xm_hbm,
              sh, deg_sh,
              xbuf, hbuf, abuf, gbuf, rbuf, sidx, didx, ones, zbuf,
              gsem, ssem, isem, dsem, nsem):
  c = lax.axis_index("c")
  s = lax.axis_index("s")
  col0 = c * DH
  r0 = s * RPT
  c0 = s * CPT
  xm = xm_hbm.at[c]
  hb = jnp.where(s < 8, r0, r0 - N_U)

  zv = jnp.zeros((16,), jnp.float32)

  def _zbuf_fill(i, _):
    zbuf[i // NSEG, pl.ds((i % NSEG) * 16, 16)] = zv
    return 0
  lax.fori_loop(0, 80 * NSEG, _zbuf_fill, 0)

  def _ones_fill(i, _):
    ones[i, :] = zv + 1.0
    return 0
  lax.fori_loop(0, ECH, _ones_fill, 0)

  def zero_rows(dst_sh):
    for off, n in _RCH:
      pltpu.sync_copy(zbuf.at[pl.ds(0, n), :],
                      dst_sh.at[pl.ds(r0 + off, n), :])

  def out_load(off, n, buf):
    @pl.when(s < 8)
    def _():
      pltpu.sync_copy(uout.at[pl.ds(hb + off, n), pl.ds(col0, DH)],
                      buf.at[pl.ds(0, n), :])

    @pl.when(s >= 8)
    def _():
      pltpu.sync_copy(iout.at[pl.ds(hb + off, n), pl.ds(col0, DH)],
                      buf.at[pl.ds(0, n), :])

  def out_store(off, n, buf):
    @pl.when(s < 8)
    def _():
      pltpu.sync_copy(buf.at[pl.ds(0, n), :],
                      uout.at[pl.ds(hb + off, n), pl.ds(col0, DH)])

    @pl.when(s >= 8)
    def _():
      pltpu.sync_copy(buf.at[pl.ds(0, n), :],
                      iout.at[pl.ds(hb + off, n), pl.ds(col0, DH)])

  zero_rows(sh)
  for off, n in _RCH:
    pltpu.sync_copy(zbuf.at[pl.ds(0, n), pl.ds(0, 16)],
                    deg_sh.at[pl.ds(r0 + off, n), :])

  def stage_load(off, n, buf):
    @pl.when(s < 8)
    def _():
      pltpu.sync_copy(user_hbm.at[pl.ds(hb + off, n), pl.ds(col0, DH)],
                      buf.at[pl.ds(0, n), :])

    @pl.when(s >= 8)
    def _():
      pltpu.sync_copy(item_hbm.at[pl.ds(hb + off, n), pl.ds(col0, DH)],
                      buf.at[pl.ds(0, n), :])

  for off, n in _RCH:
    stage_load(off, n, xbuf)
    pltpu.sync_copy(xbuf.at[pl.ds(0, n), :], xm.at[pl.ds(r0 + off, n), :])

  plsc.subcore_barrier()

  def edge_phase(count_deg):
    pltpu.sync_copy(edge_hbm.at[0, pl.ds(c0, BLK), :], sidx.at[0])
    pltpu.sync_copy(edge_hbm.at[1, pl.ds(c0, BLK), :], didx.at[0])

    def block(k, _):
      p = lax.rem(k, 2)
      pn = lax.rem(k + 1, 2)

      @pl.when(k + 1 < NBLK)
      def _():
        nb = c0 + BLK * (k + 1)
        pltpu.async_copy(edge_hbm.at[0, pl.ds(nb, BLK), :], sidx.at[pn], isem)
        pltpu.async_copy(edge_hbm.at[1, pl.ds(nb, BLK), :], didx.at[pn], isem)

      for j in range(min(NGD, BLK)):
        pltpu.async_copy(xm.at[sidx.at[p, j]], gbuf.at[j % NGB], gsem)
      for j in range(BLK):
        g = j % NGB
        if j >= NSD:
          pltpu.make_async_copy(gbuf.at[0], sh.at[didx.at[p, 0]],
                                ssem).wait()
        if j + NGD < BLK:
          pltpu.async_copy(xm.at[sidx.at[p, j + NGD]],
                           gbuf.at[(j + NGD) % NGB], gsem)
        pltpu.make_async_copy(xm.at[sidx.at[p, 0]], gbuf.at[g], gsem).wait()
        pltpu.async_copy(gbuf.at[g], sh.at[didx.at[p, j]], ssem, add=True)
        if count_deg:
          pltpu.async_copy(ones, deg_sh.at[didx.at[p, j]], dsem, add=True)

      for j in range(min(NSD, BLK)):
        pltpu.make_async_copy(gbuf.at[0], sh.at[didx.at[p, 0]], ssem).wait()
      if count_deg:
        for j in range(BLK):
          pltpu.make_async_copy(ones, deg_sh.at[didx.at[p, 0]], dsem).wait()

      @pl.when(k + 1 < NBLK)
      def _():
        pltpu.make_async_copy(edge_hbm.at[0, pl.ds(c0, BLK), :],
                              sidx.at[pn], isem).wait()
        pltpu.make_async_copy(edge_hbm.at[1, pl.ds(c0, BLK), :],
                              didx.at[pn], isem).wait()
      return 0

    lax.fori_loop(0, NBLK, block, 0)

    @pl.when(s < 4)
    def _():
      tb = N_TILES * CPT + s
      pltpu.sync_copy(edge_hbm.at[0, pl.ds(tb, 1), :], sidx.at[0, pl.ds(0, 1)])
      pltpu.sync_copy(edge_hbm.at[1, pl.ds(tb, 1), :], didx.at[0, pl.ds(0, 1)])
      pltpu.async_copy(xm.at[sidx.at[0, 0]], gbuf.at[0], gsem)
      pltpu.make_async_copy(xm.at[sidx.at[0, 0]], gbuf.at[0], gsem).wait()
      pltpu.sync_copy(gbuf.at[0], sh.at[didx.at[0, 0]], add=True)
      if count_deg:
        pltpu.sync_copy(ones, deg_sh.at[didx.at[0, 0]], add=True)

  def norm_phase(layer):
    first = layer == 0
    last = layer == NUM_LAYERS - 1
    for off, n in _RCH:
      ro = r0 + off
      pltpu.sync_copy(sh.at[pl.ds(ro, n), :], xbuf.at[pl.ds(0, n), :])
      pltpu.sync_copy(deg_sh.at[pl.ds(ro, n), :], rbuf.at[pl.ds(0, n), :])
      if first:
        pltpu.sync_copy(xm.at[pl.ds(ro, n), :], hbuf.at[pl.ds(0, n), :])
      if last:
        out_load(off, n, hbuf)
        pltpu.sync_copy(xm.at[pl.ds(ro, n), :], abuf.at[pl.ds(0, n), :])

      if first:
        @plsc.parallel_loop(0, n, unroll=4)
        def _(i):
          rbuf[i, :] = 1.0 / jnp.maximum(rbuf[i, :], 1.0)

      @plsc.parallel_loop(0, n, unroll=4)
      def _(i):
        rec = rbuf[i, :]
        for j in range(NSEG):
          sl = pl.ds(j * 16, 16)
          x = xbuf[i, sl] * rec
          if first:
            xbuf[i, sl] = x
            hbuf[i, sl] = hbuf[i, sl] + x
          elif last:
            hbuf[i, sl] = (hbuf[i, sl] + abuf[i, sl] + x) * 0.25
          else:
            xbuf[i, sl] = x

      if first:
        pltpu.sync_copy(rbuf.at[pl.ds(0, n), :], deg_sh.at[pl.ds(ro, n), :])
      if not last:
        pltpu.sync_copy(xbuf.at[pl.ds(0, n), :], xm.at[pl.ds(ro, n), :])
        pltpu.sync_copy(zbuf.at[pl.ds(0, n), :], sh.at[pl.ds(ro, n), :])
      if first or last:
        out_store(off, n, hbuf)

  for layer in range(NUM_LAYERS):
    edge_phase(layer == 0)
    plsc.subcore_barrier()
    norm_phase(layer)
    if layer < NUM_LAYERS - 1:
      plsc.subcore_barrier()


@functools.partial(
    pl.kernel,
    out_type=(
        jax.ShapeDtypeStruct((N_U, D), jnp.float32),
        jax.ShapeDtypeStruct((N_I, D), jnp.float32),
        jax.ShapeDtypeStruct((2, N_NODES, DH), jnp.float32),
    ),
    mesh=plsc.VectorSubcoreMesh(core_axis_name="c", subcore_axis_name="s"),
    compiler_params=pltpu.CompilerParams(use_tc_tiling_on_sc=False),
    scratch_types=[
        pltpu.VMEM_SHARED((N_NODES, DH), jnp.float32),
        pltpu.VMEM_SHARED((N_NODES, 16), jnp.float32),
        pltpu.VMEM((80, DH), jnp.float32),
        pltpu.VMEM((80, DH), jnp.float32),
        pltpu.VMEM((80, DH), jnp.float32),
        pltpu.VMEM((NGB, ECH, DH), jnp.float32),
        pltpu.VMEM((80, 16), jnp.float32),
        pltpu.VMEM((2, BLK, ECH), jnp.int32),
        pltpu.VMEM((2, BLK, ECH), jnp.int32),
        pltpu.VMEM((ECH, 16), jnp.float32),
        pltpu.VMEM((80, DH), jnp.float32),
        pltpu.SemaphoreType.DMA,
        pltpu.SemaphoreType.DMA,
        pltpu.SemaphoreType.DMA,
        pltpu.SemaphoreType.DMA,
        pltpu.SemaphoreType.DMA,
    ],
)
def _gcn(user_hbm, item_hbm, edge_hbm, uout, iout, xm_hbm, *scratch):
  _gcn_body(user_hbm, item_hbm, edge_hbm, uout, iout, xm_hbm, *scratch)


@jax.jit
def kernel(user_table, item_table, edge_index):
  uout, iout, _ = _gcn(user_table, item_table,
                       edge_index.reshape(2, NCHUNK, ECH))
  return uout, iout


# speedup vs baseline: 13.6850x; 1.0284x over previous
import functools

import jax
import jax.numpy as jnp
from jax import lax
from jax.experimental import pallas as pl
from jax.experimental.pallas import tpu as pltpu
from jax.experimental.pallas import tpu_sc as plsc

N_U = 5000
N_I = 5000
N_NODES = 10000
D = 128
DH = 64
E = 320000
NUM_LAYERS = 3

N_TILES = 16
RPT = N_NODES // N_TILES
ECH = 128
NCHUNK = E // ECH
CPT = NCHUNK // N_TILES
BLK = 26
NBLK = CPT // BLK
NGB = 4
NGD = 3
NSD = NGB - NGD
NSEG = DH // 16

_RCH = [(o, min(128, RPT - o)) for o in range(0, RPT, 128)]
_ZCH = [(o, min(80, RPT - o)) for o in range(0, RPT, 80)]


def _gcn_body(user_hbm, item_hbm, edge_hbm, uout, iout, xm_hbm,
              sh, deg_sh,
              xbuf, hbuf, abuf, gbuf, rbuf, sidx, didx, ones, zbuf,
              gsem, ssem, isem, dsem, nsem):
  c = lax.axis_index("c")
  s = lax.axis_index("s")
  col0 = c * DH
  r0 = s * RPT
  c0 = s * CPT
  xm = xm_hbm.at[c]
  hb = jnp.where(s < 8, r0, r0 - N_U)

  zv = jnp.zeros((16,), jnp.float32)

  def _zbuf_fill(i, _):
    zbuf[i // NSEG, pl.ds((i % NSEG) * 16, 16)] = zv
    return 0
  lax.fori_loop(0, 80 * NSEG, _zbuf_fill, 0)

  def _ones_fill(i, _):
    ones[i, :] = zv + 1.0
    return 0
  lax.fori_loop(0, ECH, _ones_fill, 0)

  def zero_rows(dst_sh):
    for off, n in _ZCH:
      pltpu.sync_copy(zbuf.at[pl.ds(0, n), :],
                      dst_sh.at[pl.ds(r0 + off, n), :])

  def out_load(off, n, buf):
    @pl.when(s < 8)
    def _():
      pltpu.sync_copy(uout.at[pl.ds(hb + off, n), pl.ds(col0, DH)],
                      buf.at[pl.ds(0, n), :])

    @pl.when(s >= 8)
    def _():
      pltpu.sync_copy(iout.at[pl.ds(hb + off, n), pl.ds(col0, DH)],
                      buf.at[pl.ds(0, n), :])

  def out_store(off, n, buf):
    @pl.when(s < 8)
    def _():
      pltpu.sync_copy(buf.at[pl.ds(0, n), :],
                      uout.at[pl.ds(hb + off, n), pl.ds(col0, DH)])

    @pl.when(s >= 8)
    def _():
      pltpu.sync_copy(buf.at[pl.ds(0, n), :],
                      iout.at[pl.ds(hb + off, n), pl.ds(col0, DH)])

  zero_rows(sh)
  for off, n in _ZCH:
    pltpu.sync_copy(zbuf.at[pl.ds(0, n), pl.ds(0, 16)],
                    deg_sh.at[pl.ds(r0 + off, n), :])

  def stage_load(off, n, buf):
    @pl.when(s < 8)
    def _():
      pltpu.sync_copy(user_hbm.at[pl.ds(hb + off, n), pl.ds(col0, DH)],
                      buf.at[pl.ds(0, n), :])

    @pl.when(s >= 8)
    def _():
      pltpu.sync_copy(item_hbm.at[pl.ds(hb + off, n), pl.ds(col0, DH)],
                      buf.at[pl.ds(0, n), :])

  for off, n in _RCH:
    stage_load(off, n, xbuf)
    pltpu.sync_copy(xbuf.at[pl.ds(0, n), :], xm.at[pl.ds(r0 + off, n), :])

  plsc.subcore_barrier()

  def edge_phase(count_deg):
    pltpu.sync_copy(edge_hbm.at[0, pl.ds(c0, BLK), :], sidx.at[0])
    pltpu.sync_copy(edge_hbm.at[1, pl.ds(c0, BLK), :], didx.at[0])

    def block(k, _):
      p = lax.rem(k, 2)
      pn = lax.rem(k + 1, 2)

      @pl.when(k + 1 < NBLK)
      def _():
        nb = c0 + BLK * (k + 1)
        pltpu.async_copy(edge_hbm.at[0, pl.ds(nb, BLK), :], sidx.at[pn], isem)
        pltpu.async_copy(edge_hbm.at[1, pl.ds(nb, BLK), :], didx.at[pn], isem)

      for j in range(min(NGD, BLK)):
        pltpu.async_copy(xm.at[sidx.at[p, j]], gbuf.at[j % NGB], gsem)
      for j in range(BLK):
        g = j % NGB
        if j >= NSD:
          pltpu.make_async_copy(gbuf.at[0], sh.at[didx.at[p, 0]],
                                ssem).wait()
        if j + NGD < BLK:
          pltpu.async_copy(xm.at[sidx.at[p, j + NGD]],
                           gbuf.at[(j + NGD) % NGB], gsem)
        pltpu.make_async_copy(xm.at[sidx.at[p, 0]], gbuf.at[g], gsem).wait()
        pltpu.async_copy(gbuf.at[g], sh.at[didx.at[p, j]], ssem, add=True)
        if count_deg:
          pltpu.async_copy(ones, deg_sh.at[didx.at[p, j]], dsem, add=True)

      for j in range(min(NSD, BLK)):
        pltpu.make_async_copy(gbuf.at[0], sh.at[didx.at[p, 0]], ssem).wait()
      if count_deg:
        for j in range(BLK):
          pltpu.make_async_copy(ones, deg_sh.at[didx.at[p, 0]], dsem).wait()

      @pl.when(k + 1 < NBLK)
      def _():
        pltpu.make_async_copy(edge_hbm.at[0, pl.ds(c0, BLK), :],
                              sidx.at[pn], isem).wait()
        pltpu.make_async_copy(edge_hbm.at[1, pl.ds(c0, BLK), :],
                              didx.at[pn], isem).wait()
      return 0

    lax.fori_loop(0, NBLK, block, 0)

    @pl.when(s < 4)
    def _():
      tb = N_TILES * CPT + s
      pltpu.sync_copy(edge_hbm.at[0, pl.ds(tb, 1), :], sidx.at[0, pl.ds(0, 1)])
      pltpu.sync_copy(edge_hbm.at[1, pl.ds(tb, 1), :], didx.at[0, pl.ds(0, 1)])
      pltpu.async_copy(xm.at[sidx.at[0, 0]], gbuf.at[0], gsem)
      pltpu.make_async_copy(xm.at[sidx.at[0, 0]], gbuf.at[0], gsem).wait()
      pltpu.sync_copy(gbuf.at[0], sh.at[didx.at[0, 0]], add=True)
      if count_deg:
        pltpu.sync_copy(ones, deg_sh.at[didx.at[0, 0]], add=True)

  def norm_phase(layer):
    first = layer == 0
    last = layer == NUM_LAYERS - 1
    for off, n in _RCH:
      ro = r0 + off
      pltpu.sync_copy(sh.at[pl.ds(ro, n), :], xbuf.at[pl.ds(0, n), :])
      pltpu.sync_copy(deg_sh.at[pl.ds(ro, n), :], rbuf.at[pl.ds(0, n), :])
      if first:
        pltpu.sync_copy(xm.at[pl.ds(ro, n), :], hbuf.at[pl.ds(0, n), :])
      if last:
        out_load(off, n, hbuf)
        pltpu.sync_copy(xm.at[pl.ds(ro, n), :], abuf.at[pl.ds(0, n), :])

      if first:
        @plsc.parallel_loop(0, n, unroll=4)
        def _(i):
          rbuf[i, :] = 1.0 / jnp.maximum(rbuf[i, :], 1.0)

      @plsc.parallel_loop(0, n, unroll=4)
      def _(i):
        rec = rbuf[i, :]
        for j in range(NSEG):
          sl = pl.ds(j * 16, 16)
          x = xbuf[i, sl] * rec
          if first:
            xbuf[i, sl] = x
            hbuf[i, sl] = hbuf[i, sl] + x
          elif last:
            hbuf[i, sl] = (hbuf[i, sl] + abuf[i, sl] + x) * 0.25
          else:
            xbuf[i, sl] = x

      if first:
        pltpu.sync_copy(rbuf.at[pl.ds(0, n), :], deg_sh.at[pl.ds(ro, n), :])
      if not last:
        pltpu.sync_copy(xbuf.at[pl.ds(0, n), :], xm.at[pl.ds(ro, n), :])
        for zo in range(0, n, 80):
          zn = min(80, n - zo)
          pltpu.sync_copy(zbuf.at[pl.ds(0, zn), :],
                          sh.at[pl.ds(ro + zo, zn), :])
      if first or last:
        out_store(off, n, hbuf)

  for layer in range(NUM_LAYERS):
    edge_phase(layer == 0)
    plsc.subcore_barrier()
    norm_phase(layer)
    if layer < NUM_LAYERS - 1:
      plsc.subcore_barrier()


@functools.partial(
    pl.kernel,
    out_type=(
        jax.ShapeDtypeStruct((N_U, D), jnp.float32),
        jax.ShapeDtypeStruct((N_I, D), jnp.float32),
        jax.ShapeDtypeStruct((2, N_NODES, DH), jnp.float32),
    ),
    mesh=plsc.VectorSubcoreMesh(core_axis_name="c", subcore_axis_name="s"),
    compiler_params=pltpu.CompilerParams(use_tc_tiling_on_sc=False),
    scratch_types=[
        pltpu.VMEM_SHARED((N_NODES, DH), jnp.float32),
        pltpu.VMEM_SHARED((N_NODES, 16), jnp.float32),
        pltpu.VMEM((128, DH), jnp.float32),
        pltpu.VMEM((128, DH), jnp.float32),
        pltpu.VMEM((128, DH), jnp.float32),
        pltpu.VMEM((NGB, ECH, DH), jnp.float32),
        pltpu.VMEM((128, 16), jnp.float32),
        pltpu.VMEM((2, BLK, ECH), jnp.int32),
        pltpu.VMEM((2, BLK, ECH), jnp.int32),
        pltpu.VMEM((ECH, 16), jnp.float32),
        pltpu.VMEM((80, DH), jnp.float32),
        pltpu.SemaphoreType.DMA,
        pltpu.SemaphoreType.DMA,
        pltpu.SemaphoreType.DMA,
        pltpu.SemaphoreType.DMA,
        pltpu.SemaphoreType.DMA,
    ],
)
def _gcn(user_hbm, item_hbm, edge_hbm, uout, iout, xm_hbm, *scratch):
  _gcn_body(user_hbm, item_hbm, edge_hbm, uout, iout, xm_hbm, *scratch)


@jax.jit
def kernel(user_table, item_table, edge_index):
  uout, iout, _ = _gcn(user_table, item_table,
                       edge_index.reshape(2, NCHUNK, ECH))
  return uout, iout


# speedup vs baseline: 13.9822x; 1.0217x over previous
import functools

import jax
import jax.numpy as jnp
from jax import lax
from jax.experimental import pallas as pl
from jax.experimental.pallas import tpu as pltpu
from jax.experimental.pallas import tpu_sc as plsc

N_U = 5000
N_I = 5000
N_NODES = 10000
D = 128
DH = 64
E = 320000
NUM_LAYERS = 3

N_TILES = 16
RPT = N_NODES // N_TILES
ECH = 128
NCHUNK = E // ECH
CPT = NCHUNK // N_TILES
BLK = 26
NBLK = CPT // BLK
NGB = 4
NGD = 3
NSD = NGB - NGD
NSEG = DH // 16

_RCH = [(o, min(128, RPT - o)) for o in range(0, RPT, 128)]
_ZCH = [(o, min(80, RPT - o)) for o in range(0, RPT, 80)]


def _gcn_body(user_hbm, item_hbm, edge_hbm, uout, iout, xm_hbm,
              sh, deg_sh,
              xbuf, hbuf, abuf, gbuf, rbuf, sidx, didx, ones, zbuf,
              gsem, ssem, isem, dsem, nsem):
  c = lax.axis_index("c")
  s = lax.axis_index("s")
  col0 = c * DH
  r0 = s * RPT
  c0 = s * CPT
  xm = xm_hbm.at[c]
  hb = jnp.where(s < 8, r0, r0 - N_U)

  zv = jnp.zeros((16,), jnp.float32)

  def _zbuf_fill(i, _):
    zbuf[i // NSEG, pl.ds((i % NSEG) * 16, 16)] = zv
    return 0
  lax.fori_loop(0, 80 * NSEG, _zbuf_fill, 0)

  def _ones_fill(i, _):
    ones[i, :] = zv + 1.0
    return 0
  lax.fori_loop(0, ECH, _ones_fill, 0)

  def zero_rows(dst_sh):
    for off, n in _ZCH:
      pltpu.sync_copy(zbuf.at[pl.ds(0, n), :],
                      dst_sh.at[pl.ds(r0 + off, n), :])

  def out_load(off, n, buf):
    @pl.when(s < 8)
    def _():
      pltpu.sync_copy(uout.at[pl.ds(hb + off, n), pl.ds(col0, DH)],
                      buf.at[pl.ds(0, n), :])

    @pl.when(s >= 8)
    def _():
      pltpu.sync_copy(iout.at[pl.ds(hb + off, n), pl.ds(col0, DH)],
                      buf.at[pl.ds(0, n), :])

  def out_store(off, n, buf):
    @pl.when(s < 8)
    def _():
      pltpu.sync_copy(buf.at[pl.ds(0, n), :],
                      uout.at[pl.ds(hb + off, n), pl.ds(col0, DH)])

    @pl.when(s >= 8)
    def _():
      pltpu.sync_copy(buf.at[pl.ds(0, n), :],
                      iout.at[pl.ds(hb + off, n), pl.ds(col0, DH)])

  zero_rows(sh)
  for off, n in _ZCH:
    pltpu.sync_copy(zbuf.at[pl.ds(0, n), pl.ds(0, 16)],
                    deg_sh.at[pl.ds(r0 + off, n), :])

  def stage_load(off, n, buf):
    @pl.when(s < 8)
    def _():
      pltpu.sync_copy(user_hbm.at[pl.ds(hb + off, n), pl.ds(col0, DH)],
                      buf.at[pl.ds(0, n), :])

    @pl.when(s >= 8)
    def _():
      pltpu.sync_copy(item_hbm.at[pl.ds(hb + off, n), pl.ds(col0, DH)],
                      buf.at[pl.ds(0, n), :])

  sbufs = [xbuf, hbuf]
  for k, (off, n) in enumerate(_RCH):
    if k >= 2:
      po, pn_ = _RCH[k - 2]
      pltpu.make_async_copy(sbufs[k % 2].at[pl.ds(0, pn_), :],
                            xm.at[pl.ds(r0 + po, pn_), :], nsem).wait()
    stage_load(off, n, sbufs[k % 2])
    pltpu.async_copy(sbufs[k % 2].at[pl.ds(0, n), :],
                     xm.at[pl.ds(r0 + off, n), :], nsem)
  for k in (len(_RCH) - 2, len(_RCH) - 1):
    po, pn_ = _RCH[k]
    pltpu.make_async_copy(sbufs[k % 2].at[pl.ds(0, pn_), :],
                          xm.at[pl.ds(r0 + po, pn_), :], nsem).wait()

  plsc.subcore_barrier()

  def edge_phase(count_deg):
    pltpu.sync_copy(edge_hbm.at[0, pl.ds(c0, BLK), :], sidx.at[0])
    pltpu.sync_copy(edge_hbm.at[1, pl.ds(c0, BLK), :], didx.at[0])

    def block(k, _):
      p = lax.rem(k, 2)
      pn = lax.rem(k + 1, 2)

      @pl.when(k + 1 < NBLK)
      def _():
        nb = c0 + BLK * (k + 1)
        pltpu.async_copy(edge_hbm.at[0, pl.ds(nb, BLK), :], sidx.at[pn], isem)
        pltpu.async_copy(edge_hbm.at[1, pl.ds(nb, BLK), :], didx.at[pn], isem)

      for j in range(min(NGD, BLK)):
        pltpu.async_copy(xm.at[sidx.at[p, j]], gbuf.at[j % NGB], gsem)
      for j in range(BLK):
        g = j % NGB
        if j >= NSD:
          pltpu.make_async_copy(gbuf.at[0], sh.at[didx.at[p, 0]],
                                ssem).wait()
        if j + NGD < BLK:
          pltpu.async_copy(xm.at[sidx.at[p, j + NGD]],
                           gbuf.at[(j + NGD) % NGB], gsem)
        pltpu.make_async_copy(xm.at[sidx.at[p, 0]], gbuf.at[g], gsem).wait()
        pltpu.async_copy(gbuf.at[g], sh.at[didx.at[p, j]], ssem, add=True)
        if count_deg:
          pltpu.async_copy(ones, deg_sh.at[didx.at[p, j]], dsem, add=True)

      for j in range(min(NSD, BLK)):
        pltpu.make_async_copy(gbuf.at[0], sh.at[didx.at[p, 0]], ssem).wait()
      if count_deg:
        for j in range(BLK):
          pltpu.make_async_copy(ones, deg_sh.at[didx.at[p, 0]], dsem).wait()

      @pl.when(k + 1 < NBLK)
      def _():
        pltpu.make_async_copy(edge_hbm.at[0, pl.ds(c0, BLK), :],
                              sidx.at[pn], isem).wait()
        pltpu.make_async_copy(edge_hbm.at[1, pl.ds(c0, BLK), :],
                              didx.at[pn], isem).wait()
      return 0

    lax.fori_loop(0, NBLK, block, 0)

    @pl.when(s < 4)
    def _():
      tb = N_TILES * CPT + s
      pltpu.sync_copy(edge_hbm.at[0, pl.ds(tb, 1), :], sidx.at[0, pl.ds(0, 1)])
      pltpu.sync_copy(edge_hbm.at[1, pl.ds(tb, 1), :], didx.at[0, pl.ds(0, 1)])
      pltpu.async_copy(xm.at[sidx.at[0, 0]], gbuf.at[0], gsem)
      pltpu.make_async_copy(xm.at[sidx.at[0, 0]], gbuf.at[0], gsem).wait()
      pltpu.sync_copy(gbuf.at[0], sh.at[didx.at[0, 0]], add=True)
      if count_deg:
        pltpu.sync_copy(ones, deg_sh.at[didx.at[0, 0]], add=True)

  def norm_phase(layer):
    first = layer == 0
    last = layer == NUM_LAYERS - 1
    for off, n in _RCH:
      ro = r0 + off
      if first:
        pltpu.async_copy(xm.at[pl.ds(ro, n), :], hbuf.at[pl.ds(0, n), :],
                         nsem)
      if last:
        pltpu.async_copy(xm.at[pl.ds(ro, n), :], abuf.at[pl.ds(0, n), :],
                         nsem)
      pltpu.sync_copy(sh.at[pl.ds(ro, n), :], xbuf.at[pl.ds(0, n), :])
      pltpu.sync_copy(deg_sh.at[pl.ds(ro, n), :], rbuf.at[pl.ds(0, n), :])
      if first:
        pltpu.make_async_copy(xm.at[pl.ds(ro, n), :],
                              hbuf.at[pl.ds(0, n), :], nsem).wait()
      if last:
        out_load(off, n, hbuf)
        pltpu.make_async_copy(xm.at[pl.ds(ro, n), :],
                              abuf.at[pl.ds(0, n), :], nsem).wait()

      if first:
        @plsc.parallel_loop(0, n, unroll=4)
        def _(i):
          rbuf[i, :] = 1.0 / jnp.maximum(rbuf[i, :], 1.0)

      @plsc.parallel_loop(0, n, unroll=4)
      def _(i):
        rec = rbuf[i, :]
        for j in range(NSEG):
          sl = pl.ds(j * 16, 16)
          x = xbuf[i, sl] * rec
          if first:
            xbuf[i, sl] = x
            hbuf[i, sl] = hbuf[i, sl] + x
          elif last:
            hbuf[i, sl] = (hbuf[i, sl] + abuf[i, sl] + x) * 0.25
          else:
            xbuf[i, sl] = x

      if first:
        pltpu.sync_copy(rbuf.at[pl.ds(0, n), :], deg_sh.at[pl.ds(ro, n), :])
      if not last:
        pltpu.sync_copy(xbuf.at[pl.ds(0, n), :], xm.at[pl.ds(ro, n), :])
        for zo in range(0, n, 80):
          zn = min(80, n - zo)
          pltpu.sync_copy(zbuf.at[pl.ds(0, zn), :],
                          sh.at[pl.ds(ro + zo, zn), :])
      if first or last:
        out_store(off, n, hbuf)

  for layer in range(NUM_LAYERS):
    edge_phase(layer == 0)
    plsc.subcore_barrier()
    norm_phase(layer)
    if layer < NUM_LAYERS - 1:
      plsc.subcore_barrier()


@functools.partial(
    pl.kernel,
    out_type=(
        jax.ShapeDtypeStruct((N_U, D), jnp.float32),
        jax.ShapeDtypeStruct((N_I, D), jnp.float32),
        jax.ShapeDtypeStruct((2, N_NODES, DH), jnp.float32),
    ),
    mesh=plsc.VectorSubcoreMesh(core_axis_name="c", subcore_axis_name="s"),
    compiler_params=pltpu.CompilerParams(use_tc_tiling_on_sc=False),
    scratch_types=[
        pltpu.VMEM_SHARED((N_NODES, DH), jnp.float32),
        pltpu.VMEM_SHARED((N_NODES, 16), jnp.float32),
        pltpu.VMEM((128, DH), jnp.float32),
        pltpu.VMEM((128, DH), jnp.float32),
        pltpu.VMEM((128, DH), jnp.float32),
        pltpu.VMEM((NGB, ECH, DH), jnp.float32),
        pltpu.VMEM((128, 16), jnp.float32),
        pltpu.VMEM((2, BLK, ECH), jnp.int32),
        pltpu.VMEM((2, BLK, ECH), jnp.int32),
        pltpu.VMEM((ECH, 16), jnp.float32),
        pltpu.VMEM((80, DH), jnp.float32),
        pltpu.SemaphoreType.DMA,
        pltpu.SemaphoreType.DMA,
        pltpu.SemaphoreType.DMA,
        pltpu.SemaphoreType.DMA,
        pltpu.SemaphoreType.DMA,
    ],
)
def _gcn(user_hbm, item_hbm, edge_hbm, uout, iout, xm_hbm, *scratch):
  _gcn_body(user_hbm, item_hbm, edge_hbm, uout, iout, xm_hbm, *scratch)


@jax.jit
def kernel(user_table, item_table, edge_index):
  uout, iout, _ = _gcn(user_table, item_table,
                       edge_index.reshape(2, NCHUNK, ECH))
  return uout, iout


# speedup vs baseline: 14.0777x; 1.0068x over previous
import functools

import jax
import jax.numpy as jnp
from jax import lax
from jax.experimental import pallas as pl
from jax.experimental.pallas import tpu as pltpu
from jax.experimental.pallas import tpu_sc as plsc

N_U = 5000
N_I = 5000
N_NODES = 10000
D = 128
DH = 64
E = 320000
NUM_LAYERS = 3

N_TILES = 16
RPT = N_NODES // N_TILES
ECH = 128
NCHUNK = E // ECH
CPT = NCHUNK // N_TILES
BLK = 26
NBLK = CPT // BLK
NGB = 4
NGD = 3
NSD = NGB - NGD
NSEG = DH // 16

_RCH = [(o, min(128, RPT - o)) for o in range(0, RPT, 128)]
_ZCH = [(o, min(80, RPT - o)) for o in range(0, RPT, 80)]


def _gcn_body(user_hbm, item_hbm, edge_hbm, uout, iout, xm_hbm,
              sh, deg_sh,
              xbuf, hbuf, abuf, gbuf, rbuf, sidx, didx, ones, zbuf,
              gsem, ssem, isem, dsem, nsem, wsem):
  c = lax.axis_index("c")
  s = lax.axis_index("s")
  col0 = c * DH
  r0 = s * RPT
  c0 = s * CPT
  xm = xm_hbm.at[c]
  hb = jnp.where(s < 8, r0, r0 - N_U)

  zv = jnp.zeros((16,), jnp.float32)

  def _zbuf_fill(i, _):
    zbuf[i // NSEG, pl.ds((i % NSEG) * 16, 16)] = zv
    return 0
  lax.fori_loop(0, 80 * NSEG, _zbuf_fill, 0)

  def _ones_fill(i, _):
    ones[i, :] = zv + 1.0
    return 0
  lax.fori_loop(0, ECH, _ones_fill, 0)

  def zero_rows(dst_sh):
    for off, n in _ZCH:
      pltpu.sync_copy(zbuf.at[pl.ds(0, n), :],
                      dst_sh.at[pl.ds(r0 + off, n), :])

  def out_load(off, n, buf):
    @pl.when(s < 8)
    def _():
      pltpu.sync_copy(uout.at[pl.ds(hb + off, n), pl.ds(col0, DH)],
                      buf.at[pl.ds(0, n), :])

    @pl.when(s >= 8)
    def _():
      pltpu.sync_copy(iout.at[pl.ds(hb + off, n), pl.ds(col0, DH)],
                      buf.at[pl.ds(0, n), :])

  def out_store(off, n, buf):
    @pl.when(s < 8)
    def _():
      pltpu.sync_copy(buf.at[pl.ds(0, n), :],
                      uout.at[pl.ds(hb + off, n), pl.ds(col0, DH)])

    @pl.when(s >= 8)
    def _():
      pltpu.sync_copy(buf.at[pl.ds(0, n), :],
                      iout.at[pl.ds(hb + off, n), pl.ds(col0, DH)])

  zero_rows(sh)
  for off, n in _ZCH:
    pltpu.sync_copy(zbuf.at[pl.ds(0, n), pl.ds(0, 16)],
                    deg_sh.at[pl.ds(r0 + off, n), :])

  def stage_load(off, n, buf):
    @pl.when(s < 8)
    def _():
      pltpu.sync_copy(user_hbm.at[pl.ds(hb + off, n), pl.ds(col0, DH)],
                      buf.at[pl.ds(0, n), :])

    @pl.when(s >= 8)
    def _():
      pltpu.sync_copy(item_hbm.at[pl.ds(hb + off, n), pl.ds(col0, DH)],
                      buf.at[pl.ds(0, n), :])

  sbufs = [xbuf, hbuf]
  for k, (off, n) in enumerate(_RCH):
    if k >= 2:
      po, pn_ = _RCH[k - 2]
      pltpu.make_async_copy(sbufs[k % 2].at[pl.ds(0, pn_), :],
                            xm.at[pl.ds(r0 + po, pn_), :], nsem).wait()
    stage_load(off, n, sbufs[k % 2])
    pltpu.async_copy(sbufs[k % 2].at[pl.ds(0, n), :],
                     xm.at[pl.ds(r0 + off, n), :], nsem)
  for k in (len(_RCH) - 2, len(_RCH) - 1):
    po, pn_ = _RCH[k]
    pltpu.make_async_copy(sbufs[k % 2].at[pl.ds(0, pn_), :],
                          xm.at[pl.ds(r0 + po, pn_), :], nsem).wait()

  plsc.subcore_barrier()

  def edge_phase(count_deg):
    pltpu.sync_copy(edge_hbm.at[0, pl.ds(c0, BLK), :], sidx.at[0])
    pltpu.sync_copy(edge_hbm.at[1, pl.ds(c0, BLK), :], didx.at[0])

    def block(k, _):
      p = lax.rem(k, 2)
      pn = lax.rem(k + 1, 2)

      @pl.when(k + 1 < NBLK)
      def _():
        nb = c0 + BLK * (k + 1)
        pltpu.async_copy(edge_hbm.at[0, pl.ds(nb, BLK), :], sidx.at[pn], isem)
        pltpu.async_copy(edge_hbm.at[1, pl.ds(nb, BLK), :], didx.at[pn], isem)

      for j in range(min(NGD, BLK)):
        pltpu.async_copy(xm.at[sidx.at[p, j]], gbuf.at[j % NGB], gsem)
      for j in range(BLK):
        g = j % NGB
        if j >= NSD:
          pltpu.make_async_copy(gbuf.at[0], sh.at[didx.at[p, 0]],
                                ssem).wait()
        if j + NGD < BLK:
          pltpu.async_copy(xm.at[sidx.at[p, j + NGD]],
                           gbuf.at[(j + NGD) % NGB], gsem)
        pltpu.make_async_copy(xm.at[sidx.at[p, 0]], gbuf.at[g], gsem).wait()
        pltpu.async_copy(gbuf.at[g], sh.at[didx.at[p, j]], ssem, add=True)
        if count_deg:
          pltpu.async_copy(ones, deg_sh.at[didx.at[p, j]], dsem, add=True)

      for j in range(min(NSD, BLK)):
        pltpu.make_async_copy(gbuf.at[0], sh.at[didx.at[p, 0]], ssem).wait()
      if count_deg:
        for j in range(BLK):
          pltpu.make_async_copy(ones, deg_sh.at[didx.at[p, 0]], dsem).wait()

      @pl.when(k + 1 < NBLK)
      def _():
        pltpu.make_async_copy(edge_hbm.at[0, pl.ds(c0, BLK), :],
                              sidx.at[pn], isem).wait()
        pltpu.make_async_copy(edge_hbm.at[1, pl.ds(c0, BLK), :],
                              didx.at[pn], isem).wait()
      return 0

    lax.fori_loop(0, NBLK, block, 0)

    @pl.when(s < 4)
    def _():
      tb = N_TILES * CPT + s
      pltpu.sync_copy(edge_hbm.at[0, pl.ds(tb, 1), :], sidx.at[0, pl.ds(0, 1)])
      pltpu.sync_copy(edge_hbm.at[1, pl.ds(tb, 1), :], didx.at[0, pl.ds(0, 1)])
      pltpu.async_copy(xm.at[sidx.at[0, 0]], gbuf.at[0], gsem)
      pltpu.make_async_copy(xm.at[sidx.at[0, 0]], gbuf.at[0], gsem).wait()
      pltpu.sync_copy(gbuf.at[0], sh.at[didx.at[0, 0]], add=True)
      if count_deg:
        pltpu.sync_copy(ones, deg_sh.at[didx.at[0, 0]], add=True)

  def norm_phase(layer):
    first = layer == 0
    last = layer == NUM_LAYERS - 1
    pend = None
    for off, n in _RCH:
      ro = r0 + off
      if first:
        pltpu.async_copy(xm.at[pl.ds(ro, n), :], hbuf.at[pl.ds(0, n), :],
                         nsem)
      if last:
        pltpu.async_copy(xm.at[pl.ds(ro, n), :], abuf.at[pl.ds(0, n), :],
                         nsem)
      pltpu.sync_copy(sh.at[pl.ds(ro, n), :], xbuf.at[pl.ds(0, n), :])
      pltpu.sync_copy(deg_sh.at[pl.ds(ro, n), :], rbuf.at[pl.ds(0, n), :])
      if pend is not None:
        pro, pn_ = pend
        pltpu.make_async_copy(abuf.at[pl.ds(0, pn_), :],
                              xm.at[pl.ds(pro, pn_), :], wsem).wait()
        pend = None
      if first:
        pltpu.make_async_copy(xm.at[pl.ds(ro, n), :],
                              hbuf.at[pl.ds(0, n), :], nsem).wait()
      if last:
        out_load(off, n, hbuf)
        pltpu.make_async_copy(xm.at[pl.ds(ro, n), :],
                              abuf.at[pl.ds(0, n), :], nsem).wait()

      if first:
        @plsc.parallel_loop(0, n, unroll=4)
        def _(i):
          rbuf[i, :] = 1.0 / jnp.maximum(rbuf[i, :], 1.0)

      @plsc.parallel_loop(0, n, unroll=4)
      def _(i):
        rec = rbuf[i, :]
        for j in range(NSEG):
          sl = pl.ds(j * 16, 16)
          x = xbuf[i, sl] * rec
          if first:
            abuf[i, sl] = x
            hbuf[i, sl] = hbuf[i, sl] + x
          elif last:
            hbuf[i, sl] = (hbuf[i, sl] + abuf[i, sl] + x) * 0.25
          else:
            abuf[i, sl] = x

      if first:
        pltpu.sync_copy(rbuf.at[pl.ds(0, n), :], deg_sh.at[pl.ds(ro, n), :])
      if not last:
        pltpu.async_copy(abuf.at[pl.ds(0, n), :], xm.at[pl.ds(ro, n), :],
                         wsem)
        pend = (ro, n)
        for zo in range(0, n, 80):
          zn = min(80, n - zo)
          pltpu.sync_copy(zbuf.at[pl.ds(0, zn), :],
                          sh.at[pl.ds(ro + zo, zn), :])
      if first or last:
        out_store(off, n, hbuf)
    if pend is not None:
      pro, pn_ = pend
      pltpu.make_async_copy(abuf.at[pl.ds(0, pn_), :],
                            xm.at[pl.ds(pro, pn_), :], wsem).wait()

  for layer in range(NUM_LAYERS):
    edge_phase(layer == 0)
    plsc.subcore_barrier()
    norm_phase(layer)
    if layer < NUM_LAYERS - 1:
      plsc.subcore_barrier()


@functools.partial(
    pl.kernel,
    out_type=(
        jax.ShapeDtypeStruct((N_U, D), jnp.float32),
        jax.ShapeDtypeStruct((N_I, D), jnp.float32),
        jax.ShapeDtypeStruct((2, N_NODES, DH), jnp.float32),
    ),
    mesh=plsc.VectorSubcoreMesh(core_axis_name="c", subcore_axis_name="s"),
    compiler_params=pltpu.CompilerParams(use_tc_tiling_on_sc=False),
    scratch_types=[
        pltpu.VMEM_SHARED((N_NODES, DH), jnp.float32),
        pltpu.VMEM_SHARED((N_NODES, 16), jnp.float32),
        pltpu.VMEM((128, DH), jnp.float32),
        pltpu.VMEM((128, DH), jnp.float32),
        pltpu.VMEM((128, DH), jnp.float32),
        pltpu.VMEM((NGB, ECH, DH), jnp.float32),
        pltpu.VMEM((128, 16), jnp.float32),
        pltpu.VMEM((2, BLK, ECH), jnp.int32),
        pltpu.VMEM((2, BLK, ECH), jnp.int32),
        pltpu.VMEM((ECH, 16), jnp.float32),
        pltpu.VMEM((80, DH), jnp.float32),
        pltpu.SemaphoreType.DMA,
        pltpu.SemaphoreType.DMA,
        pltpu.SemaphoreType.DMA,
        pltpu.SemaphoreType.DMA,
        pltpu.SemaphoreType.DMA,
        pltpu.SemaphoreType.DMA,
    ],
)
def _gcn(user_hbm, item_hbm, edge_hbm, uout, iout, xm_hbm, *scratch):
  _gcn_body(user_hbm, item_hbm, edge_hbm, uout, iout, xm_hbm, *scratch)


@jax.jit
def kernel(user_table, item_table, edge_index):
  uout, iout, _ = _gcn(user_table, item_table,
                       edge_index.reshape(2, NCHUNK, ECH))
  return uout, iout
